# Initial kernel scaffold; baseline (speedup 1.0000x reference)
import jax
import jax.numpy as jnp
from jax.experimental import pallas as pl


def kernel(z, pos, edge_index, batch, absorber_mask, params):
    raise NotImplementedError("write your pallas kernel here")


# speedup vs baseline: 41.5781x; 41.5781x over previous
import functools

import jax
import jax.numpy as jnp
from jax import lax
from jax.experimental import pallas as pl
from jax.experimental.pallas import tpu as pltpu
from jax.experimental.pallas import tpu_sc as plsc

N_NODES = 10000
N_EDGES = 160000
N_GRAPHS = 256
MUL0, MUL1, MUL2 = 64, 32, 16
HID = 240
MUL_MSG = 16
SH_DIM = 9
N_RBF = 10
R_MAX = 5.0
NUM_BASIS = 128
N_LAYERS = 4

NC, NS, LANES = 2, 16, 16
NW = NC * NS
N_PAD = 10240
E_PAD = 163840
EW = E_PAD // NW
CH = 128
NCH = EW // CH
NODES_W = N_PAD // NW
MSGW = MUL_MSG * SH_DIM

_SQ3 = 3.0 ** 0.5
_SQ15 = 15.0 ** 0.5
_SQ5H = (5.0 ** 0.5) / 2.0
_SQ15H = _SQ15 / 2.0
_RBF_W = R_MAX / (N_RBF - 1)
_RBF_C = -1.0 / (2.0 * _RBF_W * _RBF_W)

_mesh = plsc.VectorSubcoreMesh(core_axis_name="c", subcore_axis_name="s",
                               num_cores=NC, num_subcores=NS)


def _rsqrt_nr(l2):
    i = lax.bitcast_convert_type(l2, jnp.int32)
    i = jnp.int32(0x5F3759DF) - (i >> 1)
    y = lax.bitcast_convert_type(i, jnp.float32)
    for _ in range(3):
        y = y * (1.5 - 0.5 * l2 * y * y)
    return y


@functools.partial(
    pl.kernel,
    out_type=(jax.ShapeDtypeStruct((E_PAD, 16), jnp.float32),
              jax.ShapeDtypeStruct((E_PAD, 16), jnp.float32),
              jax.ShapeDtypeStruct((N_PAD, MUL0), jnp.float32)),
    mesh=_mesh,
    scratch_types=[
        pltpu.VMEM((4, 16), jnp.int32),
        pltpu.VMEM((4, 16), jnp.float32),
        pltpu.VMEM((CH,), jnp.int32),
        pltpu.VMEM((CH,), jnp.int32),
        pltpu.VMEM((CH, 16), jnp.float32),
        pltpu.VMEM((CH, 16), jnp.float32),
        pltpu.VMEM((CH, 16), jnp.float32),
        pltpu.VMEM((CH, 16), jnp.float32),
        pltpu.VMEM((64,), jnp.int32),
        pltpu.VMEM((64, MUL0), jnp.float32),
        pltpu.SemaphoreType.DMA,
        pltpu.SemaphoreType.DMA,
    ],
    compiler_params=pltpu.CompilerParams(use_tc_tiling_on_sc=False),
)
def _sc_geom(pos_hbm, src_hbm, dst_hbm, z_hbm, emb_hbm, ci_hbm, cf_hbm,
             sh_hbm, rb_hbm, h0_hbm,
             cib, cfb, sidx, didx, ps, pd, shb, rbb, zidx, embr, sem0, sem1):
    wid = lax.axis_index("c") * NS + lax.axis_index("s")
    pltpu.sync_copy(ci_hbm, cib)
    pltpu.sync_copy(cf_hbm, cfb)
    ia, ib, ic, idd = cib[0], cib[1], cib[2], cib[3]
    k1, k2, oneh3, steps = cfb[0], cfb[1], cfb[2], cfb[3]
    l0 = jnp.zeros((LANES,), jnp.int32)
    l1 = jnp.full((LANES,), 1, jnp.int32)
    l2i = jnp.full((LANES,), 2, jnp.int32)

    def _g(x, idx):
        return x.at[idx].get(mode='promise_in_bounds')

    def chunk(ci, _):
        base = wid * EW + ci * CH
        pltpu.sync_copy(src_hbm.at[pl.ds(base, CH)], sidx)
        pltpu.sync_copy(dst_hbm.at[pl.ds(base, CH)], didx)
        cps = pltpu.async_copy(pos_hbm.at[sidx], ps, sem0)
        cpd = pltpu.async_copy(pos_hbm.at[didx], pd, sem1)
        cps.wait()
        cpd.wait()

        def edge(g, _):
            for j in range(LANES):
                i = g * LANES + j
                dv = pd[i] - ps[i]
                sq = dv * dv
                l2 = _g(sq, l0) + _g(sq, l1) + _g(sq, l2i) + 1e-12
                rs = _rsqrt_nr(l2)
                t = dv * rs + oneh3
                shb[i] = k1 * _g(t, ia) * _g(t, ib) + k2 * _g(t, ic) * _g(t, idd)
                dd = l2 * rs - steps
                rbb[i] = jnp.exp(dd * dd * _RBF_C)
            return ()
        lax.fori_loop(0, CH // LANES, edge, ())
        pltpu.sync_copy(shb, sh_hbm.at[pl.ds(base, CH)])
        pltpu.sync_copy(rbb, rb_hbm.at[pl.ds(base, CH)])
        return ()

    lax.fori_loop(0, NCH, chunk, ())

    for nc in range(NODES_W // 64):
        nb = wid * NODES_W + nc * 64
        pltpu.sync_copy(z_hbm.at[pl.ds(nb, 64)], zidx)
        pltpu.async_copy(emb_hbm.at[zidx], embr, sem0).wait()
        pltpu.sync_copy(embr, h0_hbm.at[pl.ds(nb, 64)])


def _geom_consts():
    ii = [[3, 0, 1, 2, 0, 1, 2, 0, 0] + [3] * 7,
          [3, 3, 3, 3, 1, 2, 2, 2, 0] + [3] * 7,
          [3] * 6 + [3, 3, 1] + [3] * 7,
          [3] * 6 + [3, 3, 1] + [3] * 7]
    ci = jnp.array(ii, jnp.int32)
    k1 = [1.0, _SQ3, _SQ3, _SQ3, _SQ15, _SQ15, 3.0 * _SQ5H, _SQ15, _SQ15H] + [0.0] * 7
    k2 = [0.0] * 6 + [-_SQ5H, 0.0, -_SQ15H] + [0.0] * 7
    oneh3 = [0.0] * 3 + [1.0] + [0.0] * 12
    steps = [r * _RBF_W for r in range(N_RBF)] + [1e6] * 6
    cf = jnp.array([k1, k2, oneh3, steps], jnp.float32)
    return ci, cf


@functools.partial(
    pl.kernel,
    out_type=jax.ShapeDtypeStruct((NC, N_PAD, MSGW), jnp.float32),
    mesh=_mesh,
    scratch_types=[
        pltpu.VMEM((CH,), jnp.int32),
        pltpu.VMEM((CH,), jnp.int32),
        pltpu.VMEM((CH, 16), jnp.float32),
        pltpu.VMEM((CH, 16), jnp.float32),
        pltpu.VMEM((CH, 16), jnp.float32),
        pltpu.VMEM((CH, MSGW), jnp.float32),
        pltpu.VMEM((40, MSGW), jnp.float32),
        pltpu.VMEM_SHARED((N_PAD, MSGW), jnp.float32),
        pltpu.SemaphoreType.DMA,
    ],
    compiler_params=pltpu.CompilerParams(use_tc_tiling_on_sc=False),
)
def _sc_edge(src_hbm, dst_hbm, hw_hbm, rw_hbm, sh_hbm, out_hbm,
             sidx, didx, hwb, rwb, shb, msgb, zb, agg_sh, sem0):
    cid = lax.axis_index("c")
    sid = lax.axis_index("s")
    zero16 = jnp.zeros((LANES,), jnp.float32)

    def zrow(r, _):
        for cc in range(SH_DIM):
            zb[r, pl.ds(cc * 16, 16)] = zero16
        return ()
    lax.fori_loop(0, 40, zrow, ())
    for r in range(640 // 40):
        pltpu.sync_copy(zb, agg_sh.at[pl.ds(sid * 640 + r * 40, 40)])
    plsc.subcore_barrier()

    base0 = cid * (E_PAD // NC) + sid * EW

    def chunk(ci, _):
        base = base0 + ci * CH
        pltpu.sync_copy(src_hbm.at[pl.ds(base, CH)], sidx)
        pltpu.sync_copy(dst_hbm.at[pl.ds(base, CH)], didx)
        gat = pltpu.async_copy(hw_hbm.at[sidx], hwb, sem0)
        pltpu.sync_copy(rw_hbm.at[pl.ds(base, CH)], rwb)
        pltpu.sync_copy(sh_hbm.at[pl.ds(base, CH)], shb)
        gat.wait()

        def group(g, _):
            for j in range(LANES):
                i = g * LANES + j
                m = hwb[i] * rwb[i]
                shr = shb[i]
                for k in range(SH_DIM):
                    sk = shr.at[jnp.full((LANES,), k, jnp.int32)].get(
                        mode='promise_in_bounds')
                    msgb[i, pl.ds(k * 16, 16)] = m * sk
            return ()
        lax.fori_loop(0, CH // LANES, group, ())
        pltpu.sync_copy(msgb, agg_sh.at[didx], add=True)
        return ()

    lax.fori_loop(0, NCH, chunk, ())
    plsc.subcore_barrier()
    pltpu.sync_copy(agg_sh.at[pl.ds(sid * 640, 640)],
                    out_hbm.at[cid, pl.ds(sid * 640, 640)])


@functools.partial(
    pl.kernel,
    out_type=jax.ShapeDtypeStruct((N_GRAPHS, HID), jnp.float32),
    mesh=_mesh,
    scratch_types=[
        pltpu.VMEM((8,), jnp.int32),
        pltpu.VMEM((8, HID), jnp.float32),
        pltpu.SemaphoreType.DMA,
    ],
    compiler_params=pltpu.CompilerParams(use_tc_tiling_on_sc=False),
)
def _sc_gather_rows(h_hbm, idx_hbm, out_hbm, idxb, rows, sem0):
    wid = lax.axis_index("c") * NS + lax.axis_index("s")
    pltpu.sync_copy(idx_hbm.at[pl.ds(wid * 8, 8)], idxb)
    pltpu.async_copy(h_hbm.at[idxb], rows, sem0).wait()
    pltpu.sync_copy(rows, out_hbm.at[pl.ds(wid * 8, 8)])


def _silu(x):
    return x / (1.0 + jnp.exp(-x))


def _tc_radial_body(rb_ref, w1_ref, w2_ref, out_ref):
    e = pl.program_id(1)
    rb = rb_ref[...]
    t = _silu(jnp.dot(rb, w1_ref[0], preferred_element_type=jnp.float32))
    t = jnp.dot(t, w2_ref[0], preferred_element_type=jnp.float32)
    rows = lax.broadcasted_iota(jnp.int32, t.shape, 0) + e * 2048
    out_ref[0] = jnp.where(rows < N_EDGES, t, 0.0)


def _tc_radial(rb, rw1s, rw2s):
    return pl.pallas_call(
        _tc_radial_body,
        grid=(N_LAYERS, E_PAD // 2048),
        in_specs=[
            pl.BlockSpec((2048, 16), lambda l, e: (e, 0)),
            pl.BlockSpec((1, 16, 32), lambda l, e: (l, 0, 0)),
            pl.BlockSpec((1, 32, 16), lambda l, e: (l, 0, 0)),
        ],
        out_specs=pl.BlockSpec((1, 2048, 16), lambda l, e: (l, e, 0)),
        out_shape=jax.ShapeDtypeStruct((N_LAYERS, E_PAD, 16), jnp.float32),
    )(rb, rw1s, rw2s)


def _tc_hw_body(h_ref, w_ref, out_ref):
    out_ref[...] = jnp.dot(h_ref[...], w_ref[...],
                           preferred_element_type=jnp.float32)


def _tc_hw(h, w):
    return pl.pallas_call(
        _tc_hw_body,
        grid=(N_PAD // 1024,),
        in_specs=[
            pl.BlockSpec((1024, h.shape[1]), lambda i: (i, 0)),
            pl.BlockSpec(w.shape, lambda i: (0, 0)),
        ],
        out_specs=pl.BlockSpec((1024, 16), lambda i: (i, 0)),
        out_shape=jax.ShapeDtypeStruct((N_PAD, 16), jnp.float32),
    )(h, w)


def _tc_node_body(agg_ref, h_ref, wout_ref, wsc_ref, wmsg_ref, h_o, hw_o):
    a = agg_ref[0] + agg_ref[1]
    hn = (jnp.dot(a, wout_ref[...], preferred_element_type=jnp.float32)
          + jnp.dot(h_ref[...], wsc_ref[...], preferred_element_type=jnp.float32))
    h_o[...] = hn
    hw_o[...] = jnp.dot(hn, wmsg_ref[...], preferred_element_type=jnp.float32)


def _tc_node(agg2, h, wout, wsc, wmsg):
    d_in = h.shape[1]
    return pl.pallas_call(
        _tc_node_body,
        grid=(N_PAD // 1024,),
        in_specs=[
            pl.BlockSpec((NC, 1024, MSGW), lambda i: (0, i, 0)),
            pl.BlockSpec((1024, d_in), lambda i: (i, 0)),
            pl.BlockSpec((MSGW, HID), lambda i: (0, 0)),
            pl.BlockSpec((d_in, HID), lambda i: (0, 0)),
            pl.BlockSpec((HID, 16), lambda i: (0, 0)),
        ],
        out_specs=[
            pl.BlockSpec((1024, HID), lambda i: (i, 0)),
            pl.BlockSpec((1024, 16), lambda i: (i, 0)),
        ],
        out_shape=[
            jax.ShapeDtypeStruct((N_PAD, HID), jnp.float32),
            jax.ShapeDtypeStruct((N_PAD, 16), jnp.float32),
        ],
    )(agg2, h, wout, wsc, wmsg)


def _tc_readout_body(h_ref, ha_ref, ga_ref, batch_ref,
                     wq_ref, wk_ref, wv_ref,
                     w1s_ref, w1c_ref, w1v_ref, w1t_ref, b1_ref,
                     w2_ref, b2_ref, s3_ref, s5_ref, out_ref):
    scal = h_ref[:, :MUL0]
    k = jnp.dot(scal, wk_ref[...], preferred_element_type=jnp.float32)
    v = jnp.dot(scal, wv_ref[...], preferred_element_type=jnp.float32)
    sa = ha_ref[:, :MUL0]
    q = jnp.dot(sa, wq_ref[...], preferred_element_type=jnp.float32)
    scores = lax.dot_general(q, k, (((1,), (1,)), ((), ())),
                             preferred_element_type=jnp.float32)
    scores = scores * (1.0 / (MUL0 ** 0.5))
    valid = ga_ref[...] == batch_ref[...]
    scores = jnp.where(valid, scores, -1e9)
    mx = jnp.max(scores, axis=1, keepdims=True)
    e = jnp.exp(scores - mx)
    attn = e / jnp.sum(e, axis=1, keepdims=True)
    c = jnp.dot(attn, v, preferred_element_type=jnp.float32)
    vsq = ha_ref[:, MUL0:MUL0 + MUL1 * 3]
    nv = jnp.dot(vsq * vsq, s3_ref[...], preferred_element_type=jnp.float32)
    tsq = ha_ref[:, MUL0 + MUL1 * 3:HID]
    nt = jnp.dot(tsq * tsq, s5_ref[...], preferred_element_type=jnp.float32)
    zr = (jnp.dot(sa, w1s_ref[...], preferred_element_type=jnp.float32)
          + jnp.dot(c, w1c_ref[...], preferred_element_type=jnp.float32)
          + jnp.dot(nv, w1v_ref[...], preferred_element_type=jnp.float32)
          + jnp.dot(nt, w1t_ref[...], preferred_element_type=jnp.float32)
          + b1_ref[...])
    hdn = _silu(zr)
    out_ref[...] = jnp.dot(hdn, w2_ref[...],
                           preferred_element_type=jnp.float32) + b2_ref[...]


def _tc_readout(h, ha, ga2, batch2, wq, wk, wv, w1s, w1c, w1v, w1t, b1, w2, b2,
                s3, s5):
    return pl.pallas_call(
        _tc_readout_body,
        out_shape=jax.ShapeDtypeStruct((N_GRAPHS, NUM_BASIS), jnp.float32),
    )(h, ha, ga2, batch2, wq, wk, wv, w1s, w1c, w1v, w1t, b1, w2, b2, s3, s5)


def kernel(z, pos, edge_index, batch, absorber_mask, params):
    f32 = jnp.float32
    z_pad = jnp.pad(z.astype(jnp.int32), (0, N_PAD - N_NODES))
    batch_pad = jnp.pad(batch.astype(jnp.int32), (0, N_PAD - N_NODES),
                        constant_values=N_GRAPHS + 7)
    pos16 = jnp.zeros((N_PAD, 16), f32).at[:N_NODES, :3].set(pos)
    src = jnp.pad(edge_index[0].astype(jnp.int32), (0, E_PAD - N_EDGES))
    dst = jnp.pad(edge_index[1].astype(jnp.int32), (0, E_PAD - N_EDGES))
    abs_idx = jnp.nonzero(absorber_mask, size=N_GRAPHS)[0].astype(jnp.int32)
    g_a = batch[abs_idx].astype(jnp.int32)

    layers = params['layers']
    rw1s = jnp.stack([jnp.pad(lp['rw1'], ((0, 16 - N_RBF), (0, 0)))
                      for lp in layers])
    rw2s = jnp.stack([lp['rw2'] for lp in layers])
    perm = (jnp.arange(MSGW) % 16) * SH_DIM + (jnp.arange(MSGW) // 16)
    wouts = [lp['w_out'][perm] * 0.25 for lp in layers]
    s3 = (jnp.arange(MUL1 * 3)[:, None] // 3 == jnp.arange(MUL1)[None, :]).astype(f32)
    s5 = (jnp.arange(MUL2 * 5)[:, None] // 5 == jnp.arange(MUL2)[None, :]).astype(f32)
    w1 = params['w1']
    w1s, w1c = w1[:MUL0], w1[MUL0:2 * MUL0]
    w1v, w1t = w1[2 * MUL0:2 * MUL0 + MUL1], w1[2 * MUL0 + MUL1:]

    ci_const, cf_const = _geom_consts()
    sh, rb, h0 = _sc_geom(pos16, src, dst, z_pad, params['emb'], ci_const, cf_const)
    rw_all = _tc_radial(rb, rw1s, rw2s)
    h = h0
    hw = _tc_hw(h0, layers[0]['w_msg'])
    for l in range(N_LAYERS):
        agg2 = _sc_edge(src, dst, hw, rw_all[l], sh)
        wmsg_next = (layers[l + 1]['w_msg'] if l + 1 < N_LAYERS
                     else jnp.zeros((HID, 16), f32))
        h, hw = _tc_node(agg2, h, wouts[l], layers[l]['w_sc'], wmsg_next)

    ha = _sc_gather_rows(h, abs_idx)
    return _tc_readout(h, ha, g_a[:, None], batch_pad[None, :],
                       params['wq'], params['wk'], params['wv'],
                       w1s, w1c, w1v, w1t, params['b1'][None, :],
                       params['w2'], params['b2'][None, :], s3, s5)


# speedup vs baseline: 50.1466x; 1.2061x over previous
import functools

import jax
import jax.numpy as jnp
from jax import lax
from jax.experimental import pallas as pl
from jax.experimental.pallas import tpu as pltpu
from jax.experimental.pallas import tpu_sc as plsc

N_NODES = 10000
N_EDGES = 160000
N_GRAPHS = 256
MUL0, MUL1, MUL2 = 64, 32, 16
HID = 240
MUL_MSG = 16
SH_DIM = 9
N_RBF = 10
R_MAX = 5.0
NUM_BASIS = 128
N_LAYERS = 4

NC, NS, LANES = 2, 16, 16
NW = NC * NS
N_PAD = 10240
E_PAD = 163840
EW = E_PAD // NW
CH = 128
NCH = EW // CH
NODES_W = N_PAD // NW
MSGW = MUL_MSG * SH_DIM

_SQ3 = 3.0 ** 0.5
_SQ15 = 15.0 ** 0.5
_SQ5H = (5.0 ** 0.5) / 2.0
_SQ15H = _SQ15 / 2.0
_RBF_W = R_MAX / (N_RBF - 1)
_RBF_C = -1.0 / (2.0 * _RBF_W * _RBF_W)

_mesh = plsc.VectorSubcoreMesh(core_axis_name="c", subcore_axis_name="s",
                               num_cores=NC, num_subcores=NS)


def _rsqrt_nr(l2):
    i = lax.bitcast_convert_type(l2, jnp.int32)
    i = jnp.int32(0x5F3759DF) - (i >> 1)
    y = lax.bitcast_convert_type(i, jnp.float32)
    for _ in range(3):
        y = y * (1.5 - 0.5 * l2 * y * y)
    return y


@functools.partial(
    pl.kernel,
    out_type=(jax.ShapeDtypeStruct((E_PAD, 16), jnp.float32),
              jax.ShapeDtypeStruct((E_PAD, 16), jnp.float32),
              jax.ShapeDtypeStruct((N_PAD, MUL0), jnp.float32)),
    mesh=_mesh,
    scratch_types=[
        pltpu.VMEM((4, 16), jnp.int32),
        pltpu.VMEM((4, 16), jnp.float32),
        pltpu.VMEM((CH,), jnp.int32),
        pltpu.VMEM((CH,), jnp.int32),
        pltpu.VMEM((CH, 16), jnp.float32),
        pltpu.VMEM((CH, 16), jnp.float32),
        pltpu.VMEM((CH, 16), jnp.float32),
        pltpu.VMEM((CH, 16), jnp.float32),
        pltpu.VMEM((64,), jnp.int32),
        pltpu.VMEM((64, MUL0), jnp.float32),
        pltpu.SemaphoreType.DMA,
        pltpu.SemaphoreType.DMA,
    ],
    compiler_params=pltpu.CompilerParams(use_tc_tiling_on_sc=False),
)
def _sc_geom(pos_hbm, src_hbm, dst_hbm, z_hbm, emb_hbm, ci_hbm, cf_hbm,
             sh_hbm, rb_hbm, h0_hbm,
             cib, cfb, sidx, didx, ps, pd, shb, rbb, zidx, embr, sem0, sem1):
    wid = lax.axis_index("c") * NS + lax.axis_index("s")
    pltpu.sync_copy(ci_hbm, cib)
    pltpu.sync_copy(cf_hbm, cfb)
    ia, ib, ic, idd = cib[0], cib[1], cib[2], cib[3]
    k1, k2, oneh3, steps = cfb[0], cfb[1], cfb[2], cfb[3]
    l0 = jnp.zeros((LANES,), jnp.int32)
    l1 = jnp.full((LANES,), 1, jnp.int32)
    l2i = jnp.full((LANES,), 2, jnp.int32)

    def _g(x, idx):
        return x.at[idx].get(mode='promise_in_bounds')

    def chunk(ci, _):
        base = wid * EW + ci * CH
        pltpu.sync_copy(src_hbm.at[pl.ds(base, CH)], sidx)
        pltpu.sync_copy(dst_hbm.at[pl.ds(base, CH)], didx)
        cps = pltpu.async_copy(pos_hbm.at[sidx], ps, sem0)
        cpd = pltpu.async_copy(pos_hbm.at[didx], pd, sem1)
        cps.wait()
        cpd.wait()

        def edge(g, _):
            for j in range(LANES):
                i = g * LANES + j
                dv = pd[i] - ps[i]
                sq = dv * dv
                l2 = _g(sq, l0) + _g(sq, l1) + _g(sq, l2i) + 1e-12
                rs = _rsqrt_nr(l2)
                t = dv * rs + oneh3
                shb[i] = k1 * _g(t, ia) * _g(t, ib) + k2 * _g(t, ic) * _g(t, idd)
                dd = l2 * rs - steps
                rbb[i] = jnp.exp(dd * dd * _RBF_C)
            return ()
        lax.fori_loop(0, CH // LANES, edge, ())
        pltpu.sync_copy(shb, sh_hbm.at[pl.ds(base, CH)])
        pltpu.sync_copy(rbb, rb_hbm.at[pl.ds(base, CH)])
        return ()

    lax.fori_loop(0, NCH, chunk, ())

    for nc in range(NODES_W // 64):
        nb = wid * NODES_W + nc * 64
        pltpu.sync_copy(z_hbm.at[pl.ds(nb, 64)], zidx)
        pltpu.async_copy(emb_hbm.at[zidx], embr, sem0).wait()
        pltpu.sync_copy(embr, h0_hbm.at[pl.ds(nb, 64)])


def _geom_consts():
    ii = [[3, 0, 1, 2, 0, 1, 2, 0, 0] + [3] * 7,
          [3, 3, 3, 3, 1, 2, 2, 2, 0] + [3] * 7,
          [3] * 6 + [3, 3, 1] + [3] * 7,
          [3] * 6 + [3, 3, 1] + [3] * 7]
    ci = jnp.array(ii, jnp.int32)
    k1 = [1.0, _SQ3, _SQ3, _SQ3, _SQ15, _SQ15, 3.0 * _SQ5H, _SQ15, _SQ15H] + [0.0] * 7
    k2 = [0.0] * 6 + [-_SQ5H, 0.0, -_SQ15H] + [0.0] * 7
    oneh3 = [0.0] * 3 + [1.0] + [0.0] * 12
    steps = [r * _RBF_W for r in range(N_RBF)] + [1e6] * 6
    cf = jnp.array([k1, k2, oneh3, steps], jnp.float32)
    return ci, cf


NAGG = N_NODES


@functools.partial(
    pl.kernel,
    out_type=jax.ShapeDtypeStruct((NC, NAGG, MSGW), jnp.float32),
    mesh=_mesh,
    scratch_types=[
        pltpu.VMEM((NCH, CH), jnp.int32),
        pltpu.VMEM((NCH, CH), jnp.int32),
        pltpu.VMEM((2, CH, 16), jnp.float32),
        pltpu.VMEM((2, CH, 16), jnp.float32),
        pltpu.VMEM((2, CH, 16), jnp.float32),
        pltpu.VMEM((CH, MSGW), jnp.float32),
        pltpu.VMEM_SHARED((NAGG, MSGW), jnp.float32),
        pltpu.SemaphoreType.DMA,
        pltpu.SemaphoreType.DMA,
        pltpu.SemaphoreType.DMA,
        pltpu.SemaphoreType.DMA,
    ],
    compiler_params=pltpu.CompilerParams(use_tc_tiling_on_sc=False),
)
def _sc_edge(src2_hbm, dst2_hbm, hw_hbm, rw_hbm, sh_hbm, out_hbm,
             sidx, didx, hwb, rwb, shb, msgb, agg_sh,
             sg0, sg1, sl0, sl1):
    cid = lax.axis_index("c")
    sid = lax.axis_index("s")
    zero16 = jnp.zeros((LANES,), jnp.float32)
    sg = (sg0, sg1)
    sl = (sl0, sl1)

    def zmsg(r, _):
        for cc in range(SH_DIM):
            msgb[r, pl.ds(cc * 16, 16)] = zero16
        return ()
    lax.fori_loop(0, CH, zmsg, ())
    for r in range(4):
        pltpu.sync_copy(msgb, agg_sh.at[pl.ds(sid * 625 + r * CH, CH)])
    pltpu.sync_copy(msgb.at[pl.ds(0, 113)],
                    agg_sh.at[pl.ds(sid * 625 + 4 * CH, 113)])
    plsc.subcore_barrier()

    wrow0 = (cid * NS + sid) * NCH
    pltpu.sync_copy(src2_hbm.at[pl.ds(wrow0, NCH)], sidx)
    pltpu.sync_copy(dst2_hbm.at[pl.ds(wrow0, NCH)], didx)

    def issue_in(ci, b):
        g = pltpu.async_copy(hw_hbm.at[sidx.at[ci]], hwb.at[b], sg[b])
        l1 = pltpu.async_copy(rw_hbm.at[pl.ds((wrow0 + ci) * CH, CH)], rwb.at[b], sl[b])
        l2 = pltpu.async_copy(sh_hbm.at[pl.ds((wrow0 + ci) * CH, CH)], shb.at[b], sl[b])
        return g, l1, l2

    pend = [issue_in(0, 0), issue_in(1, 1)]

    def outer(go, _):
        for b in range(2):
            ci = go * 2 + b
            pltpu.make_async_copy(rw_hbm.at[pl.ds(0, CH)], hwb.at[b], sg[b]).wait()
            pltpu.make_async_copy(rw_hbm.at[pl.ds(0, CH)], rwb.at[b], sl[b]).wait()
            pltpu.make_async_copy(rw_hbm.at[pl.ds(0, CH)], shb.at[b], sl[b]).wait()

            def group(gg, _):
                for j in range(LANES):
                    i = gg * LANES + j
                    m = hwb[b, i] * rwb[b, i]
                    shr = shb[b, i]
                    for k in range(SH_DIM):
                        sk = shr.at[jnp.full((LANES,), k, jnp.int32)].get(
                            mode='promise_in_bounds')
                        msgb[i, pl.ds(k * 16, 16)] = m * sk
                return ()
            lax.fori_loop(0, CH // LANES, group, ())
            pltpu.sync_copy(msgb, agg_sh.at[didx.at[ci]], add=True)
            pf = jnp.minimum(ci + 2, NCH - 1)
            issue_in(pf, b)
        return ()

    lax.fori_loop(0, NCH // 2, outer, ())

    for b in range(2):
        g, l1, l2 = pend[b]
        g.wait()
        l1.wait()
        l2.wait()
    plsc.subcore_barrier()
    pltpu.sync_copy(agg_sh.at[pl.ds(sid * 625, 625)],
                    out_hbm.at[cid, pl.ds(sid * 625, 625)])


@functools.partial(
    pl.kernel,
    out_type=jax.ShapeDtypeStruct((N_GRAPHS, HID), jnp.float32),
    mesh=_mesh,
    scratch_types=[
        pltpu.VMEM((8,), jnp.int32),
        pltpu.VMEM((8, HID), jnp.float32),
        pltpu.SemaphoreType.DMA,
    ],
    compiler_params=pltpu.CompilerParams(use_tc_tiling_on_sc=False),
)
def _sc_gather_rows(h_hbm, idx_hbm, out_hbm, idxb, rows, sem0):
    wid = lax.axis_index("c") * NS + lax.axis_index("s")
    pltpu.sync_copy(idx_hbm.at[pl.ds(wid * 8, 8)], idxb)
    pltpu.async_copy(h_hbm.at[idxb], rows, sem0).wait()
    pltpu.sync_copy(rows, out_hbm.at[pl.ds(wid * 8, 8)])


def _silu(x):
    return x / (1.0 + jnp.exp(-x))


def _tc_radial_body(rb_ref, w1_ref, w2_ref, out_ref):
    e = pl.program_id(1)
    rb = rb_ref[...]
    t = _silu(jnp.dot(rb, w1_ref[0], preferred_element_type=jnp.float32))
    t = jnp.dot(t, w2_ref[0], preferred_element_type=jnp.float32)
    rows = lax.broadcasted_iota(jnp.int32, t.shape, 0) + e * 2048
    out_ref[0] = jnp.where(rows < N_EDGES, t, 0.0)


def _tc_radial(rb, rw1s, rw2s):
    return pl.pallas_call(
        _tc_radial_body,
        grid=(N_LAYERS, E_PAD // 2048),
        in_specs=[
            pl.BlockSpec((2048, 16), lambda l, e: (e, 0)),
            pl.BlockSpec((1, 16, 32), lambda l, e: (l, 0, 0)),
            pl.BlockSpec((1, 32, 16), lambda l, e: (l, 0, 0)),
        ],
        out_specs=pl.BlockSpec((1, 2048, 16), lambda l, e: (l, e, 0)),
        out_shape=jax.ShapeDtypeStruct((N_LAYERS, E_PAD, 16), jnp.float32),
    )(rb, rw1s, rw2s)


def _tc_hw_body(h_ref, w_ref, out_ref):
    out_ref[...] = jnp.dot(h_ref[...], w_ref[...],
                           preferred_element_type=jnp.float32)


def _tc_hw(h, w):
    n = h.shape[0]
    return pl.pallas_call(
        _tc_hw_body,
        grid=(n // 1000,),
        in_specs=[
            pl.BlockSpec((1000, h.shape[1]), lambda i: (i, 0)),
            pl.BlockSpec(w.shape, lambda i: (0, 0)),
        ],
        out_specs=pl.BlockSpec((1000, 16), lambda i: (i, 0)),
        out_shape=jax.ShapeDtypeStruct((n, 16), jnp.float32),
    )(h, w)


def _tc_node_body(agg_ref, h_ref, wout_ref, wsc_ref, wmsg_ref, h_o, hw_o):
    a = agg_ref[0] + agg_ref[1]
    hn = (jnp.dot(a, wout_ref[...], preferred_element_type=jnp.float32)
          + jnp.dot(h_ref[...], wsc_ref[...], preferred_element_type=jnp.float32))
    h_o[...] = hn
    hw_o[...] = jnp.dot(hn, wmsg_ref[...], preferred_element_type=jnp.float32)


def _tc_node(agg2, h, wout, wsc, wmsg):
    d_in = h.shape[1]
    return pl.pallas_call(
        _tc_node_body,
        grid=(N_NODES // 1000,),
        in_specs=[
            pl.BlockSpec((NC, 1000, MSGW), lambda i: (0, i, 0)),
            pl.BlockSpec((1000, d_in), lambda i: (i, 0)),
            pl.BlockSpec((MSGW, HID), lambda i: (0, 0)),
            pl.BlockSpec((d_in, HID), lambda i: (0, 0)),
            pl.BlockSpec((HID, 16), lambda i: (0, 0)),
        ],
        out_specs=[
            pl.BlockSpec((1000, HID), lambda i: (i, 0)),
            pl.BlockSpec((1000, 16), lambda i: (i, 0)),
        ],
        out_shape=[
            jax.ShapeDtypeStruct((N_NODES, HID), jnp.float32),
            jax.ShapeDtypeStruct((N_NODES, 16), jnp.float32),
        ],
    )(agg2, h, wout, wsc, wmsg)


def _tc_readout_body(h_ref, ha_ref, ga_ref, batch_ref,
                     wq_ref, wk_ref, wv_ref,
                     w1s_ref, w1c_ref, w1v_ref, w1t_ref, b1_ref,
                     w2_ref, b2_ref, s3_ref, s5_ref, out_ref):
    scal = h_ref[:, :MUL0]
    k = jnp.dot(scal, wk_ref[...], preferred_element_type=jnp.float32)
    v = jnp.dot(scal, wv_ref[...], preferred_element_type=jnp.float32)
    sa = ha_ref[:, :MUL0]
    q = jnp.dot(sa, wq_ref[...], preferred_element_type=jnp.float32)
    scores = lax.dot_general(q, k, (((1,), (1,)), ((), ())),
                             preferred_element_type=jnp.float32)
    scores = scores * (1.0 / (MUL0 ** 0.5))
    valid = ga_ref[...] == batch_ref[...]
    scores = jnp.where(valid, scores, -1e9)
    mx = jnp.max(scores, axis=1, keepdims=True)
    e = jnp.exp(scores - mx)
    attn = e / jnp.sum(e, axis=1, keepdims=True)
    c = jnp.dot(attn, v, preferred_element_type=jnp.float32)
    vsq = ha_ref[:, MUL0:MUL0 + MUL1 * 3]
    nv = jnp.dot(vsq * vsq, s3_ref[...], preferred_element_type=jnp.float32)
    tsq = ha_ref[:, MUL0 + MUL1 * 3:HID]
    nt = jnp.dot(tsq * tsq, s5_ref[...], preferred_element_type=jnp.float32)
    zr = (jnp.dot(sa, w1s_ref[...], preferred_element_type=jnp.float32)
          + jnp.dot(c, w1c_ref[...], preferred_element_type=jnp.float32)
          + jnp.dot(nv, w1v_ref[...], preferred_element_type=jnp.float32)
          + jnp.dot(nt, w1t_ref[...], preferred_element_type=jnp.float32)
          + b1_ref[...])
    hdn = _silu(zr)
    out_ref[...] = jnp.dot(hdn, w2_ref[...],
                           preferred_element_type=jnp.float32) + b2_ref[...]


def _tc_readout(h, ha, ga2, batch2, wq, wk, wv, w1s, w1c, w1v, w1t, b1, w2, b2,
                s3, s5):
    return pl.pallas_call(
        _tc_readout_body,
        out_shape=jax.ShapeDtypeStruct((N_GRAPHS, NUM_BASIS), jnp.float32),
    )(h, ha, ga2, batch2, wq, wk, wv, w1s, w1c, w1v, w1t, b1, w2, b2, s3, s5)


def kernel(z, pos, edge_index, batch, absorber_mask, params):
    f32 = jnp.float32
    z_pad = jnp.pad(z.astype(jnp.int32), (0, N_PAD - N_NODES))
    pos16 = jnp.zeros((N_PAD, 16), f32).at[:N_NODES, :3].set(pos)
    src = jnp.pad(edge_index[0].astype(jnp.int32), (0, E_PAD - N_EDGES))
    dst = jnp.pad(edge_index[1].astype(jnp.int32), (0, E_PAD - N_EDGES))
    abs_idx = jnp.nonzero(absorber_mask, size=N_GRAPHS)[0].astype(jnp.int32)
    g_a = batch[abs_idx].astype(jnp.int32)

    layers = params['layers']
    rw1s = jnp.stack([jnp.pad(lp['rw1'], ((0, 16 - N_RBF), (0, 0)))
                      for lp in layers])
    rw2s = jnp.stack([lp['rw2'] for lp in layers])
    perm = (jnp.arange(MSGW) % 16) * SH_DIM + (jnp.arange(MSGW) // 16)
    wouts = [lp['w_out'][perm] * 0.25 for lp in layers]
    s3 = (jnp.arange(MUL1 * 3)[:, None] // 3 == jnp.arange(MUL1)[None, :]).astype(f32)
    s5 = (jnp.arange(MUL2 * 5)[:, None] // 5 == jnp.arange(MUL2)[None, :]).astype(f32)
    w1 = params['w1']
    w1s, w1c = w1[:MUL0], w1[MUL0:2 * MUL0]
    w1v, w1t = w1[2 * MUL0:2 * MUL0 + MUL1], w1[2 * MUL0 + MUL1:]

    ci_const, cf_const = _geom_consts()
    src2 = src.reshape(E_PAD // CH, CH)
    dst2 = dst.reshape(E_PAD // CH, CH)
    sh, rb, h0 = _sc_geom(pos16, src, dst, z_pad, params['emb'], ci_const, cf_const)
    rw_all = _tc_radial(rb, rw1s, rw2s)
    h = h0[:N_NODES]
    hw = _tc_hw(h, layers[0]['w_msg'])
    for l in range(N_LAYERS):
        agg2 = _sc_edge(src2, dst2, hw, rw_all[l], sh)
        wmsg_next = (layers[l + 1]['w_msg'] if l + 1 < N_LAYERS
                     else jnp.zeros((HID, 16), f32))
        h, hw = _tc_node(agg2, h, wouts[l], layers[l]['w_sc'], wmsg_next)

    ha = _sc_gather_rows(h, abs_idx)
    return _tc_readout(h, ha, g_a[:, None], batch.astype(jnp.int32)[None, :],
                       params['wq'], params['wk'], params['wv'],
                       w1s, w1c, w1v, w1t, params['b1'][None, :],
                       params['w2'], params['b2'][None, :], s3, s5)


# speedup vs baseline: 53.1430x; 1.0598x over previous
import functools

import jax
import jax.numpy as jnp
from jax import lax
from jax.experimental import pallas as pl
from jax.experimental.pallas import tpu as pltpu
from jax.experimental.pallas import tpu_sc as plsc

N_NODES = 10000
N_EDGES = 160000
N_GRAPHS = 256
MUL0, MUL1, MUL2 = 64, 32, 16
HID = 240
MUL_MSG = 16
SH_DIM = 9
N_RBF = 10
R_MAX = 5.0
NUM_BASIS = 128
N_LAYERS = 4

NC, NS, LANES = 2, 16, 16
NW = NC * NS
N_PAD = 10240
E_PAD = 163840
EW = E_PAD // NW
CH = 128
NCH = EW // CH
NODES_W = N_PAD // NW
MSGW = MUL_MSG * SH_DIM

_SQ3 = 3.0 ** 0.5
_SQ15 = 15.0 ** 0.5
_SQ5H = (5.0 ** 0.5) / 2.0
_SQ15H = _SQ15 / 2.0
_RBF_W = R_MAX / (N_RBF - 1)
_RBF_C = -1.0 / (2.0 * _RBF_W * _RBF_W)

_mesh = plsc.VectorSubcoreMesh(core_axis_name="c", subcore_axis_name="s",
                               num_cores=NC, num_subcores=NS)


def _rsqrt_nr(l2):
    i = lax.bitcast_convert_type(l2, jnp.int32)
    i = jnp.int32(0x5F3759DF) - (i >> 1)
    y = lax.bitcast_convert_type(i, jnp.float32)
    for _ in range(3):
        y = y * (1.5 - 0.5 * l2 * y * y)
    return y


@functools.partial(
    pl.kernel,
    out_type=(jax.ShapeDtypeStruct((E_PAD, 16), jnp.float32),
              jax.ShapeDtypeStruct((E_PAD, 16), jnp.float32),
              jax.ShapeDtypeStruct((N_PAD, MUL0), jnp.float32)),
    mesh=_mesh,
    scratch_types=[
        pltpu.VMEM((4, 16), jnp.int32),
        pltpu.VMEM((4, 16), jnp.float32),
        pltpu.VMEM((NCH, CH), jnp.int32),
        pltpu.VMEM((NCH, CH), jnp.int32),
        pltpu.VMEM((2, CH, 16), jnp.float32),
        pltpu.VMEM((2, CH, 16), jnp.float32),
        pltpu.VMEM((2, CH, 16), jnp.float32),
        pltpu.VMEM((2, CH, 16), jnp.float32),
        pltpu.VMEM((64,), jnp.int32),
        pltpu.VMEM((64, MUL0), jnp.float32),
        pltpu.SemaphoreType.DMA,
        pltpu.SemaphoreType.DMA,
        pltpu.SemaphoreType.DMA,
        pltpu.SemaphoreType.DMA,
        pltpu.SemaphoreType.DMA,
        pltpu.SemaphoreType.DMA,
    ],
    compiler_params=pltpu.CompilerParams(use_tc_tiling_on_sc=False),
)
def _sc_geom(pos_hbm, src2_hbm, dst2_hbm, z_hbm, emb_hbm, ci_hbm, cf_hbm,
             sh_hbm, rb_hbm, h0_hbm,
             cib, cfb, sidx, didx, ps, pd, shb, rbb, zidx, embr,
             sa0, sa1, sb0, sb1, so0, so1):
    wid = lax.axis_index("c") * NS + lax.axis_index("s")
    pltpu.sync_copy(ci_hbm, cib)
    pltpu.sync_copy(cf_hbm, cfb)
    ia, ib, ic, idd = cib[0], cib[1], cib[2], cib[3]
    k1, k2, oneh3, steps = cfb[0], cfb[1], cfb[2], cfb[3]
    l0 = jnp.zeros((LANES,), jnp.int32)
    l1v = jnp.full((LANES,), 1, jnp.int32)
    l2i = jnp.full((LANES,), 2, jnp.int32)
    sa = (sa0, sa1)
    sb = (sb0, sb1)
    so = (so0, so1)
    wrow0 = wid * NCH

    def _g(x, idx):
        return x.at[idx].get(mode='promise_in_bounds')

    pltpu.sync_copy(src2_hbm.at[pl.ds(wrow0, NCH)], sidx)
    pltpu.sync_copy(dst2_hbm.at[pl.ds(wrow0, NCH)], didx)

    def issue_in(ci, b):
        g1 = pltpu.async_copy(pos_hbm.at[sidx.at[ci]], ps.at[b], sa[b])
        g2 = pltpu.async_copy(pos_hbm.at[didx.at[ci]], pd.at[b], sb[b])
        return g1, g2

    pend = [issue_in(0, 0), issue_in(1, 1)]

    def chunk(ci, _):
        for b in range(2):
            cc = ci * 2 + b
            base = (wrow0 + cc) * CH
            pltpu.make_async_copy(sh_hbm.at[pl.ds(0, CH)], ps.at[b], sa[b]).wait()
            pltpu.make_async_copy(sh_hbm.at[pl.ds(0, CH)], pd.at[b], sb[b]).wait()

            @pl.when(cc >= 2)
            def _():
                pltpu.make_async_copy(shb.at[b], sh_hbm.at[pl.ds(0, CH)], so[b]).wait()
                pltpu.make_async_copy(rbb.at[b], rb_hbm.at[pl.ds(0, CH)], so[b]).wait()

            def edge(g, _):
                for j in range(LANES):
                    i = g * LANES + j
                    dv = pd[b, i] - ps[b, i]
                    sq = dv * dv
                    l2 = _g(sq, l0) + _g(sq, l1v) + _g(sq, l2i) + 1e-12
                    rs = _rsqrt_nr(l2)
                    t = dv * rs + oneh3
                    shb[b, i] = (k1 * _g(t, ia) * _g(t, ib)
                                 + k2 * _g(t, ic) * _g(t, idd))
                    dd = l2 * rs - steps
                    rbb[b, i] = jnp.exp(dd * dd * _RBF_C)
                return ()
            lax.fori_loop(0, CH // LANES, edge, ())
            pltpu.async_copy(shb.at[b], sh_hbm.at[pl.ds(base, CH)], so[b])
            pltpu.async_copy(rbb.at[b], rb_hbm.at[pl.ds(base, CH)], so[b])
            pf = jnp.minimum(cc + 2, NCH - 1)
            issue_in(pf, b)
        return ()

    lax.fori_loop(0, NCH // 2, chunk, ())

    for b in range(2):
        g1, g2 = pend[b]
        g1.wait()
        g2.wait()
        pltpu.make_async_copy(shb.at[b], sh_hbm.at[pl.ds(0, CH)], so[b]).wait()
        pltpu.make_async_copy(rbb.at[b], rb_hbm.at[pl.ds(0, CH)], so[b]).wait()

    for nc in range(NODES_W // 64):
        nb = wid * NODES_W + nc * 64
        pltpu.sync_copy(z_hbm.at[pl.ds(nb, 64)], zidx)
        pltpu.async_copy(emb_hbm.at[zidx], embr, sa0).wait()
        pltpu.sync_copy(embr, h0_hbm.at[pl.ds(nb, 64)])


def _geom_consts():
    ii = [[3, 0, 1, 2, 0, 1, 2, 0, 0] + [3] * 7,
          [3, 3, 3, 3, 1, 2, 2, 2, 0] + [3] * 7,
          [3] * 6 + [3, 3, 1] + [3] * 7,
          [3] * 6 + [3, 3, 1] + [3] * 7]
    ci = jnp.array(ii, jnp.int32)
    k1 = [1.0, _SQ3, _SQ3, _SQ3, _SQ15, _SQ15, 3.0 * _SQ5H, _SQ15, _SQ15H] + [0.0] * 7
    k2 = [0.0] * 6 + [-_SQ5H, 0.0, -_SQ15H] + [0.0] * 7
    oneh3 = [0.0] * 3 + [1.0] + [0.0] * 12
    steps = [r * _RBF_W for r in range(N_RBF)] + [1e6] * 6
    cf = jnp.array([k1, k2, oneh3, steps], jnp.float32)
    return ci, cf


NAGG = N_NODES


@functools.partial(
    pl.kernel,
    out_type=jax.ShapeDtypeStruct((NC, NAGG, MSGW), jnp.float32),
    mesh=_mesh,
    scratch_types=[
        pltpu.VMEM((NCH, CH), jnp.int32),
        pltpu.VMEM((NCH, CH), jnp.int32),
        pltpu.VMEM((2, CH, 16), jnp.float32),
        pltpu.VMEM((2, CH, 16), jnp.float32),
        pltpu.VMEM((2, CH, 16), jnp.float32),
        pltpu.VMEM((CH, MSGW), jnp.float32),
        pltpu.VMEM_SHARED((NAGG, MSGW), jnp.float32),
        pltpu.SemaphoreType.DMA,
        pltpu.SemaphoreType.DMA,
        pltpu.SemaphoreType.DMA,
        pltpu.SemaphoreType.DMA,
    ],
    compiler_params=pltpu.CompilerParams(use_tc_tiling_on_sc=False),
)
def _sc_edge(src2_hbm, dst2_hbm, hw_hbm, rw_hbm, sh_hbm, out_hbm,
             sidx, didx, hwb, rwb, shb, msgb, agg_sh,
             sg0, sg1, sl0, sl1):
    cid = lax.axis_index("c")
    sid = lax.axis_index("s")
    zero16 = jnp.zeros((LANES,), jnp.float32)
    sg = (sg0, sg1)
    sl = (sl0, sl1)

    def zmsg(r, _):
        for cc in range(SH_DIM):
            msgb[r, pl.ds(cc * 16, 16)] = zero16
        return ()
    lax.fori_loop(0, CH, zmsg, ())
    for r in range(4):
        pltpu.sync_copy(msgb, agg_sh.at[pl.ds(sid * 625 + r * CH, CH)])
    pltpu.sync_copy(msgb.at[pl.ds(0, 113)],
                    agg_sh.at[pl.ds(sid * 625 + 4 * CH, 113)])
    plsc.subcore_barrier()

    wrow0 = (cid * NS + sid) * NCH
    pltpu.sync_copy(src2_hbm.at[pl.ds(wrow0, NCH)], sidx)
    pltpu.sync_copy(dst2_hbm.at[pl.ds(wrow0, NCH)], didx)

    def issue_in(ci, b):
        g = pltpu.async_copy(hw_hbm.at[sidx.at[ci]], hwb.at[b], sg[b])
        l1 = pltpu.async_copy(rw_hbm.at[pl.ds((wrow0 + ci) * CH, CH)], rwb.at[b], sl[b])
        l2 = pltpu.async_copy(sh_hbm.at[pl.ds((wrow0 + ci) * CH, CH)], shb.at[b], sl[b])
        return g, l1, l2

    pend = [issue_in(0, 0), issue_in(1, 1)]

    def outer(go, _):
        for b in range(2):
            ci = go * 2 + b
            pltpu.make_async_copy(rw_hbm.at[pl.ds(0, CH)], hwb.at[b], sg[b]).wait()
            pltpu.make_async_copy(rw_hbm.at[pl.ds(0, CH)], rwb.at[b], sl[b]).wait()
            pltpu.make_async_copy(rw_hbm.at[pl.ds(0, CH)], shb.at[b], sl[b]).wait()

            def group(gg, _):
                for j in range(LANES):
                    i = gg * LANES + j
                    m = hwb[b, i] * rwb[b, i]
                    shr = shb[b, i]
                    for k in range(SH_DIM):
                        sk = shr.at[jnp.full((LANES,), k, jnp.int32)].get(
                            mode='promise_in_bounds')
                        msgb[i, pl.ds(k * 16, 16)] = m * sk
                return ()
            lax.fori_loop(0, CH // LANES, group, ())
            pltpu.sync_copy(msgb, agg_sh.at[didx.at[ci]], add=True)
            pf = jnp.minimum(ci + 2, NCH - 1)
            issue_in(pf, b)
        return ()

    lax.fori_loop(0, NCH // 2, outer, ())

    for b in range(2):
        g, l1, l2 = pend[b]
        g.wait()
        l1.wait()
        l2.wait()
    plsc.subcore_barrier()
    pltpu.sync_copy(agg_sh.at[pl.ds(sid * 625, 625)],
                    out_hbm.at[cid, pl.ds(sid * 625, 625)])


@functools.partial(
    pl.kernel,
    out_type=jax.ShapeDtypeStruct((N_GRAPHS, HID), jnp.float32),
    mesh=_mesh,
    scratch_types=[
        pltpu.VMEM((8,), jnp.int32),
        pltpu.VMEM((8, HID), jnp.float32),
        pltpu.SemaphoreType.DMA,
    ],
    compiler_params=pltpu.CompilerParams(use_tc_tiling_on_sc=False),
)
def _sc_gather_rows(h_hbm, idx_hbm, out_hbm, idxb, rows, sem0):
    wid = lax.axis_index("c") * NS + lax.axis_index("s")
    pltpu.sync_copy(idx_hbm.at[pl.ds(wid * 8, 8)], idxb)
    pltpu.async_copy(h_hbm.at[idxb], rows, sem0).wait()
    pltpu.sync_copy(rows, out_hbm.at[pl.ds(wid * 8, 8)])


def _silu(x):
    return x / (1.0 + jnp.exp(-x))


def _tc_radial_body(rb_ref, w1_ref, w2_ref, out_ref):
    e = pl.program_id(1)
    rb = rb_ref[...]
    t = _silu(jnp.dot(rb, w1_ref[0], preferred_element_type=jnp.float32))
    t = jnp.dot(t, w2_ref[0], preferred_element_type=jnp.float32)
    rows = lax.broadcasted_iota(jnp.int32, t.shape, 0) + e * 2048
    out_ref[0] = jnp.where(rows < N_EDGES, t, 0.0)


def _tc_radial(rb, rw1s, rw2s):
    return pl.pallas_call(
        _tc_radial_body,
        grid=(N_LAYERS, E_PAD // 2048),
        in_specs=[
            pl.BlockSpec((2048, 16), lambda l, e: (e, 0)),
            pl.BlockSpec((1, 16, 32), lambda l, e: (l, 0, 0)),
            pl.BlockSpec((1, 32, 16), lambda l, e: (l, 0, 0)),
        ],
        out_specs=pl.BlockSpec((1, 2048, 16), lambda l, e: (l, e, 0)),
        out_shape=jax.ShapeDtypeStruct((N_LAYERS, E_PAD, 16), jnp.float32),
    )(rb, rw1s, rw2s)


def _tc_hw_body(h_ref, w_ref, out_ref):
    out_ref[...] = jnp.dot(h_ref[...], w_ref[...],
                           preferred_element_type=jnp.float32)


def _tc_hw(h, w):
    n = h.shape[0]
    return pl.pallas_call(
        _tc_hw_body,
        grid=(n // 1000,),
        in_specs=[
            pl.BlockSpec((1000, h.shape[1]), lambda i: (i, 0)),
            pl.BlockSpec(w.shape, lambda i: (0, 0)),
        ],
        out_specs=pl.BlockSpec((1000, 16), lambda i: (i, 0)),
        out_shape=jax.ShapeDtypeStruct((n, 16), jnp.float32),
    )(h, w)


def _tc_node_body(agg_ref, h_ref, wout_ref, wsc_ref, wmsg_ref, h_o, hw_o):
    a = agg_ref[0] + agg_ref[1]
    hn = (jnp.dot(a, wout_ref[...], preferred_element_type=jnp.float32)
          + jnp.dot(h_ref[...], wsc_ref[...], preferred_element_type=jnp.float32))
    h_o[...] = hn
    hw_o[...] = jnp.dot(hn, wmsg_ref[...], preferred_element_type=jnp.float32)


def _tc_node(agg2, h, wout, wsc, wmsg):
    d_in = h.shape[1]
    return pl.pallas_call(
        _tc_node_body,
        grid=(N_NODES // 1000,),
        in_specs=[
            pl.BlockSpec((NC, 1000, MSGW), lambda i: (0, i, 0)),
            pl.BlockSpec((1000, d_in), lambda i: (i, 0)),
            pl.BlockSpec((MSGW, HID), lambda i: (0, 0)),
            pl.BlockSpec((d_in, HID), lambda i: (0, 0)),
            pl.BlockSpec((HID, 16), lambda i: (0, 0)),
        ],
        out_specs=[
            pl.BlockSpec((1000, HID), lambda i: (i, 0)),
            pl.BlockSpec((1000, 16), lambda i: (i, 0)),
        ],
        out_shape=[
            jax.ShapeDtypeStruct((N_NODES, HID), jnp.float32),
            jax.ShapeDtypeStruct((N_NODES, 16), jnp.float32),
        ],
    )(agg2, h, wout, wsc, wmsg)


def _tc_readout_body(h_ref, ha_ref, ga_ref, batch_ref,
                     wq_ref, wk_ref, wv_ref,
                     w1s_ref, w1c_ref, w1v_ref, w1t_ref, b1_ref,
                     w2_ref, b2_ref, s3_ref, s5_ref, out_ref):
    scal = h_ref[:, :MUL0]
    k = jnp.dot(scal, wk_ref[...], preferred_element_type=jnp.float32)
    v = jnp.dot(scal, wv_ref[...], preferred_element_type=jnp.float32)
    sa = ha_ref[:, :MUL0]
    q = jnp.dot(sa, wq_ref[...], preferred_element_type=jnp.float32)
    scores = lax.dot_general(q, k, (((1,), (1,)), ((), ())),
                             preferred_element_type=jnp.float32)
    scores = scores * (1.0 / (MUL0 ** 0.5))
    valid = ga_ref[...] == batch_ref[...]
    scores = jnp.where(valid, scores, -1e9)
    mx = jnp.max(scores, axis=1, keepdims=True)
    e = jnp.exp(scores - mx)
    attn = e / jnp.sum(e, axis=1, keepdims=True)
    c = jnp.dot(attn, v, preferred_element_type=jnp.float32)
    vsq = ha_ref[:, MUL0:MUL0 + MUL1 * 3]
    nv = jnp.dot(vsq * vsq, s3_ref[...], preferred_element_type=jnp.float32)
    tsq = ha_ref[:, MUL0 + MUL1 * 3:HID]
    nt = jnp.dot(tsq * tsq, s5_ref[...], preferred_element_type=jnp.float32)
    zr = (jnp.dot(sa, w1s_ref[...], preferred_element_type=jnp.float32)
          + jnp.dot(c, w1c_ref[...], preferred_element_type=jnp.float32)
          + jnp.dot(nv, w1v_ref[...], preferred_element_type=jnp.float32)
          + jnp.dot(nt, w1t_ref[...], preferred_element_type=jnp.float32)
          + b1_ref[...])
    hdn = _silu(zr)
    out_ref[...] = jnp.dot(hdn, w2_ref[...],
                           preferred_element_type=jnp.float32) + b2_ref[...]


def _tc_readout(h, ha, ga2, batch2, wq, wk, wv, w1s, w1c, w1v, w1t, b1, w2, b2,
                s3, s5):
    return pl.pallas_call(
        _tc_readout_body,
        out_shape=jax.ShapeDtypeStruct((N_GRAPHS, NUM_BASIS), jnp.float32),
    )(h, ha, ga2, batch2, wq, wk, wv, w1s, w1c, w1v, w1t, b1, w2, b2, s3, s5)


def kernel(z, pos, edge_index, batch, absorber_mask, params):
    f32 = jnp.float32
    z_pad = jnp.pad(z.astype(jnp.int32), (0, N_PAD - N_NODES))
    pos16 = jnp.zeros((N_PAD, 16), f32).at[:N_NODES, :3].set(pos)
    src = jnp.pad(edge_index[0].astype(jnp.int32), (0, E_PAD - N_EDGES))
    dst = jnp.pad(edge_index[1].astype(jnp.int32), (0, E_PAD - N_EDGES))
    abs_idx = jnp.nonzero(absorber_mask, size=N_GRAPHS)[0].astype(jnp.int32)
    g_a = batch[abs_idx].astype(jnp.int32)

    layers = params['layers']
    rw1s = jnp.stack([jnp.pad(lp['rw1'], ((0, 16 - N_RBF), (0, 0)))
                      for lp in layers])
    rw2s = jnp.stack([lp['rw2'] for lp in layers])
    perm = (jnp.arange(MSGW) % 16) * SH_DIM + (jnp.arange(MSGW) // 16)
    wouts = [lp['w_out'][perm] * 0.25 for lp in layers]
    s3 = (jnp.arange(MUL1 * 3)[:, None] // 3 == jnp.arange(MUL1)[None, :]).astype(f32)
    s5 = (jnp.arange(MUL2 * 5)[:, None] // 5 == jnp.arange(MUL2)[None, :]).astype(f32)
    w1 = params['w1']
    w1s, w1c = w1[:MUL0], w1[MUL0:2 * MUL0]
    w1v, w1t = w1[2 * MUL0:2 * MUL0 + MUL1], w1[2 * MUL0 + MUL1:]

    ci_const, cf_const = _geom_consts()
    src2 = src.reshape(E_PAD // CH, CH)
    dst2 = dst.reshape(E_PAD // CH, CH)
    sh, rb, h0 = _sc_geom(pos16, src2, dst2, z_pad, params['emb'], ci_const, cf_const)
    rw_all = _tc_radial(rb, rw1s, rw2s)
    h = h0[:N_NODES]
    hw = _tc_hw(h, layers[0]['w_msg'])
    for l in range(N_LAYERS):
        agg2 = _sc_edge(src2, dst2, hw, rw_all[l], sh)
        wmsg_next = (layers[l + 1]['w_msg'] if l + 1 < N_LAYERS
                     else jnp.zeros((HID, 16), f32))
        h, hw = _tc_node(agg2, h, wouts[l], layers[l]['w_sc'], wmsg_next)

    ha = _sc_gather_rows(h, abs_idx)
    return _tc_readout(h, ha, g_a[:, None], batch.astype(jnp.int32)[None, :],
                       params['wq'], params['wk'], params['wv'],
                       w1s, w1c, w1v, w1t, params['b1'][None, :],
                       params['w2'], params['b2'][None, :], s3, s5)


# speedup vs baseline: 76.8418x; 1.4459x over previous
import functools

import jax
import jax.numpy as jnp
from jax import lax
from jax.experimental import pallas as pl
from jax.experimental.pallas import tpu as pltpu
from jax.experimental.pallas import tpu_sc as plsc

N_NODES = 10000
N_EDGES = 160000
N_GRAPHS = 256
MUL0, MUL1, MUL2 = 64, 32, 16
HID = 240
MUL_MSG = 16
SH_DIM = 9
N_RBF = 10
R_MAX = 5.0
NUM_BASIS = 128
N_LAYERS = 4

NC, NS, LANES = 2, 16, 16
NW = NC * NS
N_PAD = 10240
E_PAD = 163840
EW = E_PAD // NW
CH = 128
NCH = EW // CH
NODES_W = N_PAD // NW
MSGW = MUL_MSG * SH_DIM

_SQ3 = 3.0 ** 0.5
_SQ15 = 15.0 ** 0.5
_SQ5H = (5.0 ** 0.5) / 2.0
_SQ15H = _SQ15 / 2.0
_RBF_W = R_MAX / (N_RBF - 1)
_RBF_C = -1.0 / (2.0 * _RBF_W * _RBF_W)

_mesh = plsc.VectorSubcoreMesh(core_axis_name="c", subcore_axis_name="s",
                               num_cores=NC, num_subcores=NS)


def _rsqrt_nr(l2):
    i = lax.bitcast_convert_type(l2, jnp.int32)
    i = jnp.int32(0x5F3759DF) - (i >> 1)
    y = lax.bitcast_convert_type(i, jnp.float32)
    for _ in range(3):
        y = y * (1.5 - 0.5 * l2 * y * y)
    return y


@functools.partial(
    pl.kernel,
    out_type=(jax.ShapeDtypeStruct((E_PAD, 16), jnp.float32),
              jax.ShapeDtypeStruct((E_PAD // 8, 128), jnp.float32),
              jax.ShapeDtypeStruct((N_PAD, MUL0), jnp.float32)),
    mesh=_mesh,
    scratch_types=[
        pltpu.VMEM((4, 16), jnp.int32),
        pltpu.VMEM((4, 16), jnp.float32),
        pltpu.VMEM((NCH, CH), jnp.int32),
        pltpu.VMEM((NCH, CH), jnp.int32),
        pltpu.VMEM((2, CH, 16), jnp.float32),
        pltpu.VMEM((2, CH, 16), jnp.float32),
        pltpu.VMEM((2, CH, 16), jnp.float32),
        pltpu.VMEM((2, CH // 8, 128), jnp.float32),
        pltpu.VMEM((64,), jnp.int32),
        pltpu.VMEM((64, MUL0), jnp.float32),
        pltpu.SemaphoreType.DMA,
        pltpu.SemaphoreType.DMA,
        pltpu.SemaphoreType.DMA,
        pltpu.SemaphoreType.DMA,
        pltpu.SemaphoreType.DMA,
        pltpu.SemaphoreType.DMA,
    ],
    compiler_params=pltpu.CompilerParams(use_tc_tiling_on_sc=False),
)
def _sc_geom(pos_hbm, src2_hbm, dst2_hbm, z_hbm, emb_hbm, ci_hbm, cf_hbm,
             sh_hbm, rb_hbm, h0_hbm,
             cib, cfb, sidx, didx, ps, pd, shb, rbb, zidx, embr,
             sa0, sa1, sb0, sb1, so0, so1):
    wid = lax.axis_index("c") * NS + lax.axis_index("s")
    pltpu.sync_copy(ci_hbm, cib)
    pltpu.sync_copy(cf_hbm, cfb)
    ia, ib, ic, idd = cib[0], cib[1], cib[2], cib[3]
    k1, k2, oneh3, steps = cfb[0], cfb[1], cfb[2], cfb[3]
    l0 = jnp.zeros((LANES,), jnp.int32)
    l1v = jnp.full((LANES,), 1, jnp.int32)
    l2i = jnp.full((LANES,), 2, jnp.int32)
    sa = (sa0, sa1)
    sb = (sb0, sb1)
    so = (so0, so1)
    wrow0 = wid * NCH

    def _g(x, idx):
        return x.at[idx].get(mode='promise_in_bounds')

    pltpu.sync_copy(src2_hbm.at[pl.ds(wrow0, NCH)], sidx)
    pltpu.sync_copy(dst2_hbm.at[pl.ds(wrow0, NCH)], didx)

    def issue_in(ci, b):
        g1 = pltpu.async_copy(pos_hbm.at[sidx.at[ci]], ps.at[b], sa[b])
        g2 = pltpu.async_copy(pos_hbm.at[didx.at[ci]], pd.at[b], sb[b])
        return g1, g2

    pend = [issue_in(0, 0), issue_in(1, 1)]

    def chunk(ci, _):
        for b in range(2):
            cc = ci * 2 + b
            base = (wrow0 + cc) * CH
            pltpu.make_async_copy(sh_hbm.at[pl.ds(0, CH)], ps.at[b], sa[b]).wait()
            pltpu.make_async_copy(sh_hbm.at[pl.ds(0, CH)], pd.at[b], sb[b]).wait()

            @pl.when(cc >= 2)
            def _():
                pltpu.make_async_copy(shb.at[b], sh_hbm.at[pl.ds(0, CH)], so[b]).wait()
                pltpu.make_async_copy(rbb.at[b], rb_hbm.at[pl.ds(0, CH // 8)], so[b]).wait()

            def edge(g, _):
                for j in range(LANES):
                    i = g * LANES + j
                    dv = pd[b, i] - ps[b, i]
                    sq = dv * dv
                    l2 = _g(sq, l0) + _g(sq, l1v) + _g(sq, l2i) + 1e-12
                    rs = _rsqrt_nr(l2)
                    t = dv * rs + oneh3
                    shb[b, i] = (k1 * _g(t, ia) * _g(t, ib)
                                 + k2 * _g(t, ic) * _g(t, idd))
                    dd = l2 * rs - steps
                    rbb[b, 2 * g + j // 8, pl.ds((j % 8) * 16, 16)] = (
                        jnp.exp(dd * dd * _RBF_C))
                return ()
            lax.fori_loop(0, CH // LANES, edge, ())
            pltpu.async_copy(shb.at[b], sh_hbm.at[pl.ds(base, CH)], so[b])
            pltpu.async_copy(rbb.at[b], rb_hbm.at[pl.ds(base // 8, CH // 8)], so[b])
            pf = jnp.minimum(cc + 2, NCH - 1)
            issue_in(pf, b)
        return ()

    lax.fori_loop(0, NCH // 2, chunk, ())

    for b in range(2):
        g1, g2 = pend[b]
        g1.wait()
        g2.wait()
        pltpu.make_async_copy(shb.at[b], sh_hbm.at[pl.ds(0, CH)], so[b]).wait()
        pltpu.make_async_copy(rbb.at[b], rb_hbm.at[pl.ds(0, CH // 8)], so[b]).wait()

    for nc in range(NODES_W // 64):
        nb = wid * NODES_W + nc * 64
        pltpu.sync_copy(z_hbm.at[pl.ds(nb, 64)], zidx)
        pltpu.async_copy(emb_hbm.at[zidx], embr, sa0).wait()
        pltpu.sync_copy(embr, h0_hbm.at[pl.ds(nb, 64)])


def _geom_consts():
    ii = [[3, 0, 1, 2, 0, 1, 2, 0, 0] + [3] * 7,
          [3, 3, 3, 3, 1, 2, 2, 2, 0] + [3] * 7,
          [3] * 6 + [3, 3, 1] + [3] * 7,
          [3] * 6 + [3, 3, 1] + [3] * 7]
    ci = jnp.array(ii, jnp.int32)
    k1 = [1.0, _SQ3, _SQ3, _SQ3, _SQ15, _SQ15, 3.0 * _SQ5H, _SQ15, _SQ15H] + [0.0] * 7
    k2 = [0.0] * 6 + [-_SQ5H, 0.0, -_SQ15H] + [0.0] * 7
    oneh3 = [0.0] * 3 + [1.0] + [0.0] * 12
    steps = [r * _RBF_W for r in range(N_RBF)] + [1e6] * 6
    cf = jnp.array([k1, k2, oneh3, steps], jnp.float32)
    return ci, cf


NAGG = N_NODES


@functools.partial(
    pl.kernel,
    out_type=jax.ShapeDtypeStruct((NC, NAGG, MSGW), jnp.float32),
    mesh=_mesh,
    scratch_types=[
        pltpu.VMEM((NCH, CH), jnp.int32),
        pltpu.VMEM((NCH, CH), jnp.int32),
        pltpu.VMEM((2, CH, 16), jnp.float32),
        pltpu.VMEM((2, CH // 8, 128), jnp.float32),
        pltpu.VMEM((2, CH, 16), jnp.float32),
        pltpu.VMEM((CH, MSGW), jnp.float32),
        pltpu.VMEM_SHARED((NAGG, MSGW), jnp.float32),
        pltpu.SemaphoreType.DMA,
        pltpu.SemaphoreType.DMA,
        pltpu.SemaphoreType.DMA,
        pltpu.SemaphoreType.DMA,
    ],
    compiler_params=pltpu.CompilerParams(use_tc_tiling_on_sc=False),
)
def _sc_edge(src2_hbm, dst2_hbm, hw_hbm, rw_hbm, sh_hbm, out_hbm,
             sidx, didx, hwb, rwb, shb, msgb, agg_sh,
             sg0, sg1, sl0, sl1):
    cid = lax.axis_index("c")
    sid = lax.axis_index("s")
    zero16 = jnp.zeros((LANES,), jnp.float32)
    sg = (sg0, sg1)
    sl = (sl0, sl1)

    def zmsg(r, _):
        for cc in range(SH_DIM):
            msgb[r, pl.ds(cc * 16, 16)] = zero16
        return ()
    lax.fori_loop(0, CH, zmsg, ())
    for r in range(4):
        pltpu.sync_copy(msgb, agg_sh.at[pl.ds(sid * 625 + r * CH, CH)])
    pltpu.sync_copy(msgb.at[pl.ds(0, 113)],
                    agg_sh.at[pl.ds(sid * 625 + 4 * CH, 113)])
    plsc.subcore_barrier()

    wrow0 = (cid * NS + sid) * NCH
    pltpu.sync_copy(src2_hbm.at[pl.ds(wrow0, NCH)], sidx)
    pltpu.sync_copy(dst2_hbm.at[pl.ds(wrow0, NCH)], didx)

    def issue_in(ci, b):
        g = pltpu.async_copy(hw_hbm.at[sidx.at[ci]], hwb.at[b], sg[b])
        l1 = pltpu.async_copy(rw_hbm.at[pl.ds((wrow0 + ci) * (CH // 8), CH // 8)], rwb.at[b], sl[b])
        l2 = pltpu.async_copy(sh_hbm.at[pl.ds((wrow0 + ci) * CH, CH)], shb.at[b], sl[b])
        return g, l1, l2

    pend = [issue_in(0, 0), issue_in(1, 1)]

    def outer(go, _):
        for b in range(2):
            ci = go * 2 + b
            pltpu.make_async_copy(rw_hbm.at[pl.ds(0, CH)], hwb.at[b], sg[b]).wait()
            pltpu.make_async_copy(rw_hbm.at[pl.ds(0, CH // 8)], rwb.at[b], sl[b]).wait()
            pltpu.make_async_copy(rw_hbm.at[pl.ds(0, CH)], shb.at[b], sl[b]).wait()

            def group(gg, _):
                for j in range(LANES):
                    i = gg * LANES + j
                    m = hwb[b, i] * rwb[b, 2 * gg + j // 8, pl.ds((j % 8) * 16, 16)]
                    shr = shb[b, i]
                    for k in range(SH_DIM):
                        sk = shr.at[jnp.full((LANES,), k, jnp.int32)].get(
                            mode='promise_in_bounds')
                        msgb[i, pl.ds(k * 16, 16)] = m * sk
                return ()
            lax.fori_loop(0, CH // LANES, group, ())
            pltpu.sync_copy(msgb, agg_sh.at[didx.at[ci]], add=True)
            pf = jnp.minimum(ci + 2, NCH - 1)
            issue_in(pf, b)
        return ()

    lax.fori_loop(0, NCH // 2, outer, ())

    for b in range(2):
        g, l1, l2 = pend[b]
        g.wait()
        l1.wait()
        l2.wait()
    plsc.subcore_barrier()
    pltpu.sync_copy(agg_sh.at[pl.ds(sid * 625, 625)],
                    out_hbm.at[cid, pl.ds(sid * 625, 625)])


@functools.partial(
    pl.kernel,
    out_type=jax.ShapeDtypeStruct((N_GRAPHS, HID), jnp.float32),
    mesh=_mesh,
    scratch_types=[
        pltpu.VMEM((8,), jnp.int32),
        pltpu.VMEM((8, HID), jnp.float32),
        pltpu.SemaphoreType.DMA,
    ],
    compiler_params=pltpu.CompilerParams(use_tc_tiling_on_sc=False),
)
def _sc_gather_rows(h_hbm, idx_hbm, out_hbm, idxb, rows, sem0):
    wid = lax.axis_index("c") * NS + lax.axis_index("s")
    pltpu.sync_copy(idx_hbm.at[pl.ds(wid * 8, 8)], idxb)
    pltpu.async_copy(h_hbm.at[idxb], rows, sem0).wait()
    pltpu.sync_copy(rows, out_hbm.at[pl.ds(wid * 8, 8)])


def _silu(x):
    return x / (1.0 + jnp.exp(-x))


def _tc_radial_body(rb_ref, w1_ref, w2_ref, out_ref):
    e = pl.program_id(0)
    rb = rb_ref[...]
    rows = lax.broadcasted_iota(jnp.int32, (256, 1), 0) + e * 256
    msk = (rows < N_EDGES * 16 // 128).astype(jnp.float32)
    pieces = [[] for _ in range(N_LAYERS)]
    for g in range(8):
        t = _silu(jnp.dot(rb[:, g * 16:(g + 1) * 16], w1_ref[...],
                          preferred_element_type=jnp.float32))
        for l in range(N_LAYERS):
            pieces[l].append(jnp.dot(t[:, l * 32:(l + 1) * 32], w2_ref[l],
                                     preferred_element_type=jnp.float32))
    for l in range(N_LAYERS):
        out_ref[l] = jnp.concatenate(pieces[l], axis=1) * msk


def _tc_radial(rb, rw1cat, rw2s):
    return pl.pallas_call(
        _tc_radial_body,
        grid=(E_PAD // 8 // 256,),
        in_specs=[
            pl.BlockSpec((256, 128), lambda i: (i, 0)),
            pl.BlockSpec((16, 128), lambda i: (0, 0)),
            pl.BlockSpec((N_LAYERS, 32, 16), lambda i: (0, 0, 0)),
        ],
        out_specs=pl.BlockSpec((N_LAYERS, 256, 128), lambda i: (0, i, 0)),
        out_shape=jax.ShapeDtypeStruct((N_LAYERS, E_PAD // 8, 128), jnp.float32),
    )(rb, rw1cat, rw2s)


def _tc_hw_body(h_ref, w_ref, out_ref):
    out_ref[...] = jnp.dot(h_ref[...], w_ref[...],
                           preferred_element_type=jnp.float32)


def _tc_hw(h, w):
    n = h.shape[0]
    return pl.pallas_call(
        _tc_hw_body,
        grid=(n // 1000,),
        in_specs=[
            pl.BlockSpec((1000, h.shape[1]), lambda i: (i, 0)),
            pl.BlockSpec(w.shape, lambda i: (0, 0)),
        ],
        out_specs=pl.BlockSpec((1000, 16), lambda i: (i, 0)),
        out_shape=jax.ShapeDtypeStruct((n, 16), jnp.float32),
    )(h, w)


def _tc_node_body(agg_ref, h_ref, wout_ref, wsc_ref, wmsg_ref, h_o, hw_o):
    a = agg_ref[0] + agg_ref[1]
    hn = (jnp.dot(a, wout_ref[...], preferred_element_type=jnp.float32)
          + jnp.dot(h_ref[...], wsc_ref[...], preferred_element_type=jnp.float32))
    h_o[...] = hn
    hw_o[...] = jnp.dot(hn, wmsg_ref[...], preferred_element_type=jnp.float32)


def _tc_node(agg2, h, wout, wsc, wmsg):
    d_in = h.shape[1]
    return pl.pallas_call(
        _tc_node_body,
        grid=(N_NODES // 1000,),
        in_specs=[
            pl.BlockSpec((NC, 1000, MSGW), lambda i: (0, i, 0)),
            pl.BlockSpec((1000, d_in), lambda i: (i, 0)),
            pl.BlockSpec((MSGW, HID), lambda i: (0, 0)),
            pl.BlockSpec((d_in, HID), lambda i: (0, 0)),
            pl.BlockSpec((HID, 16), lambda i: (0, 0)),
        ],
        out_specs=[
            pl.BlockSpec((1000, HID), lambda i: (i, 0)),
            pl.BlockSpec((1000, 16), lambda i: (i, 0)),
        ],
        out_shape=[
            jax.ShapeDtypeStruct((N_NODES, HID), jnp.float32),
            jax.ShapeDtypeStruct((N_NODES, 16), jnp.float32),
        ],
    )(agg2, h, wout, wsc, wmsg)


def _tc_readout_body(h_ref, ha_ref, ga_ref, batch_ref,
                     wq_ref, wk_ref, wv_ref,
                     w1s_ref, w1c_ref, w1v_ref, w1t_ref, b1_ref,
                     w2_ref, b2_ref, s3_ref, s5_ref, out_ref):
    scal = h_ref[:, :MUL0]
    k = jnp.dot(scal, wk_ref[...], preferred_element_type=jnp.float32)
    v = jnp.dot(scal, wv_ref[...], preferred_element_type=jnp.float32)
    sa = ha_ref[:, :MUL0]
    q = jnp.dot(sa, wq_ref[...], preferred_element_type=jnp.float32)
    scores = lax.dot_general(q, k, (((1,), (1,)), ((), ())),
                             preferred_element_type=jnp.float32)
    scores = scores * (1.0 / (MUL0 ** 0.5))
    valid = ga_ref[...] == batch_ref[...]
    scores = jnp.where(valid, scores, -1e9)
    mx = jnp.max(scores, axis=1, keepdims=True)
    e = jnp.exp(scores - mx)
    attn = e / jnp.sum(e, axis=1, keepdims=True)
    c = jnp.dot(attn, v, preferred_element_type=jnp.float32)
    vsq = ha_ref[:, MUL0:MUL0 + MUL1 * 3]
    nv = jnp.dot(vsq * vsq, s3_ref[...], preferred_element_type=jnp.float32)
    tsq = ha_ref[:, MUL0 + MUL1 * 3:HID]
    nt = jnp.dot(tsq * tsq, s5_ref[...], preferred_element_type=jnp.float32)
    zr = (jnp.dot(sa, w1s_ref[...], preferred_element_type=jnp.float32)
          + jnp.dot(c, w1c_ref[...], preferred_element_type=jnp.float32)
          + jnp.dot(nv, w1v_ref[...], preferred_element_type=jnp.float32)
          + jnp.dot(nt, w1t_ref[...], preferred_element_type=jnp.float32)
          + b1_ref[...])
    hdn = _silu(zr)
    out_ref[...] = jnp.dot(hdn, w2_ref[...],
                           preferred_element_type=jnp.float32) + b2_ref[...]


def _tc_readout(h, ha, ga2, batch2, wq, wk, wv, w1s, w1c, w1v, w1t, b1, w2, b2,
                s3, s5):
    return pl.pallas_call(
        _tc_readout_body,
        out_shape=jax.ShapeDtypeStruct((N_GRAPHS, NUM_BASIS), jnp.float32),
    )(h, ha, ga2, batch2, wq, wk, wv, w1s, w1c, w1v, w1t, b1, w2, b2, s3, s5)


def kernel(z, pos, edge_index, batch, absorber_mask, params):
    f32 = jnp.float32
    z_pad = jnp.pad(z.astype(jnp.int32), (0, N_PAD - N_NODES))
    pos16 = jnp.zeros((N_PAD, 16), f32).at[:N_NODES, :3].set(pos)
    src = jnp.pad(edge_index[0].astype(jnp.int32), (0, E_PAD - N_EDGES))
    dst = jnp.pad(edge_index[1].astype(jnp.int32), (0, E_PAD - N_EDGES))
    abs_idx = jnp.nonzero(absorber_mask, size=N_GRAPHS)[0].astype(jnp.int32)
    g_a = batch[abs_idx].astype(jnp.int32)

    layers = params['layers']
    rw1cat = jnp.concatenate(
        [jnp.pad(lp['rw1'], ((0, 16 - N_RBF), (0, 0))) for lp in layers], axis=1)
    rw2s = jnp.stack([lp['rw2'] for lp in layers])
    perm = (jnp.arange(MSGW) % 16) * SH_DIM + (jnp.arange(MSGW) // 16)
    wouts = [lp['w_out'][perm] * 0.25 for lp in layers]
    s3 = (jnp.arange(MUL1 * 3)[:, None] // 3 == jnp.arange(MUL1)[None, :]).astype(f32)
    s5 = (jnp.arange(MUL2 * 5)[:, None] // 5 == jnp.arange(MUL2)[None, :]).astype(f32)
    w1 = params['w1']
    w1s, w1c = w1[:MUL0], w1[MUL0:2 * MUL0]
    w1v, w1t = w1[2 * MUL0:2 * MUL0 + MUL1], w1[2 * MUL0 + MUL1:]

    ci_const, cf_const = _geom_consts()
    src2 = src.reshape(E_PAD // CH, CH)
    dst2 = dst.reshape(E_PAD // CH, CH)
    sh, rb, h0 = _sc_geom(pos16, src2, dst2, z_pad, params['emb'], ci_const, cf_const)
    rw_all = _tc_radial(rb, rw1cat, rw2s)
    h = h0[:N_NODES]
    hw = _tc_hw(h, layers[0]['w_msg'])
    for l in range(N_LAYERS):
        agg2 = _sc_edge(src2, dst2, hw, rw_all[l], sh)
        wmsg_next = (layers[l + 1]['w_msg'] if l + 1 < N_LAYERS
                     else jnp.zeros((HID, 16), f32))
        h, hw = _tc_node(agg2, h, wouts[l], layers[l]['w_sc'], wmsg_next)

    ha = _sc_gather_rows(h, abs_idx)
    return _tc_readout(h, ha, g_a[:, None], batch.astype(jnp.int32)[None, :],
                       params['wq'], params['wk'], params['wv'],
                       w1s, w1c, w1v, w1t, params['b1'][None, :],
                       params['w2'], params['b2'][None, :], s3, s5)


# speedup vs baseline: 85.1259x; 1.1078x over previous
import functools

import jax
import jax.numpy as jnp
from jax import lax
from jax.experimental import pallas as pl
from jax.experimental.pallas import tpu as pltpu
from jax.experimental.pallas import tpu_sc as plsc

N_NODES = 10000
N_EDGES = 160000
N_GRAPHS = 256
MUL0, MUL1, MUL2 = 64, 32, 16
HID = 240
MUL_MSG = 16
SH_DIM = 9
N_RBF = 10
R_MAX = 5.0
NUM_BASIS = 128
N_LAYERS = 4

NC, NS, LANES = 2, 16, 16
NW = NC * NS
N_PAD = 10240
E_PAD = 163840
EW = E_PAD // NW
CH = 80
NCH = EW // CH
NODES_W = N_PAD // NW
MSGW = MUL_MSG * SH_DIM

_SQ3 = 3.0 ** 0.5
_SQ15 = 15.0 ** 0.5
_SQ5H = (5.0 ** 0.5) / 2.0
_SQ15H = _SQ15 / 2.0
_RBF_W = R_MAX / (N_RBF - 1)
_RBF_C = -1.0 / (2.0 * _RBF_W * _RBF_W)

_mesh = plsc.VectorSubcoreMesh(core_axis_name="c", subcore_axis_name="s",
                               num_cores=NC, num_subcores=NS)


def _rsqrt_nr(l2):
    i = lax.bitcast_convert_type(l2, jnp.int32)
    i = jnp.int32(0x5F3759DF) - (i >> 1)
    y = lax.bitcast_convert_type(i, jnp.float32)
    for _ in range(3):
        y = y * (1.5 - 0.5 * l2 * y * y)
    return y


@functools.partial(
    pl.kernel,
    out_type=(jax.ShapeDtypeStruct((E_PAD, 16), jnp.float32),
              jax.ShapeDtypeStruct((E_PAD // 8, 128), jnp.float32),
              jax.ShapeDtypeStruct((N_PAD, MUL0), jnp.float32)),
    mesh=_mesh,
    scratch_types=[
        pltpu.VMEM((4, 16), jnp.int32),
        pltpu.VMEM((4, 16), jnp.float32),
        pltpu.VMEM((NCH, CH), jnp.int32),
        pltpu.VMEM((NCH, CH), jnp.int32),
        pltpu.VMEM((2, CH, 16), jnp.float32),
        pltpu.VMEM((2, CH, 16), jnp.float32),
        pltpu.VMEM((2, CH, 16), jnp.float32),
        pltpu.VMEM((2, CH // 8, 128), jnp.float32),
        pltpu.VMEM((64,), jnp.int32),
        pltpu.VMEM((64, MUL0), jnp.float32),
        pltpu.SemaphoreType.DMA,
        pltpu.SemaphoreType.DMA,
        pltpu.SemaphoreType.DMA,
        pltpu.SemaphoreType.DMA,
        pltpu.SemaphoreType.DMA,
        pltpu.SemaphoreType.DMA,
    ],
    compiler_params=pltpu.CompilerParams(use_tc_tiling_on_sc=False),
)
def _sc_geom(pos_hbm, src2_hbm, dst2_hbm, z_hbm, emb_hbm, ci_hbm, cf_hbm,
             sh_hbm, rb_hbm, h0_hbm,
             cib, cfb, sidx, didx, ps, pd, shb, rbb, zidx, embr,
             sa0, sa1, sb0, sb1, so0, so1):
    wid = lax.axis_index("c") * NS + lax.axis_index("s")
    pltpu.sync_copy(ci_hbm, cib)
    pltpu.sync_copy(cf_hbm, cfb)
    ia, ib, ic, idd = cib[0], cib[1], cib[2], cib[3]
    k1, k2, oneh3, steps = cfb[0], cfb[1], cfb[2], cfb[3]
    l0 = jnp.zeros((LANES,), jnp.int32)
    l1v = jnp.full((LANES,), 1, jnp.int32)
    l2i = jnp.full((LANES,), 2, jnp.int32)
    sa = (sa0, sa1)
    sb = (sb0, sb1)
    so = (so0, so1)
    wrow0 = wid * NCH

    def _g(x, idx):
        return x.at[idx].get(mode='promise_in_bounds')

    pltpu.sync_copy(src2_hbm.at[pl.ds(wrow0, NCH)], sidx)
    pltpu.sync_copy(dst2_hbm.at[pl.ds(wrow0, NCH)], didx)

    def issue_in(ci, b):
        g1 = pltpu.async_copy(pos_hbm.at[sidx.at[ci]], ps.at[b], sa[b])
        g2 = pltpu.async_copy(pos_hbm.at[didx.at[ci]], pd.at[b], sb[b])
        return g1, g2

    pend = [issue_in(0, 0), issue_in(1, 1)]

    def chunk(ci, _):
        for b in range(2):
            cc = ci * 2 + b
            base = (wrow0 + cc) * CH
            pltpu.make_async_copy(sh_hbm.at[pl.ds(0, CH)], ps.at[b], sa[b]).wait()
            pltpu.make_async_copy(sh_hbm.at[pl.ds(0, CH)], pd.at[b], sb[b]).wait()

            @pl.when(cc >= 2)
            def _():
                pltpu.make_async_copy(shb.at[b], sh_hbm.at[pl.ds(0, CH)], so[b]).wait()
                pltpu.make_async_copy(rbb.at[b], rb_hbm.at[pl.ds(0, CH // 8)], so[b]).wait()

            def edge(g, _):
                for j in range(LANES):
                    i = g * LANES + j
                    dv = pd[b, i] - ps[b, i]
                    sq = dv * dv
                    l2 = _g(sq, l0) + _g(sq, l1v) + _g(sq, l2i) + 1e-12
                    rs = _rsqrt_nr(l2)
                    t = dv * rs + oneh3
                    shb[b, i] = (k1 * _g(t, ia) * _g(t, ib)
                                 + k2 * _g(t, ic) * _g(t, idd))
                    dd = l2 * rs - steps
                    rbb[b, 2 * g + j // 8, pl.ds((j % 8) * 16, 16)] = (
                        jnp.exp(dd * dd * _RBF_C))
                return ()
            lax.fori_loop(0, CH // LANES, edge, ())
            pltpu.async_copy(shb.at[b], sh_hbm.at[pl.ds(base, CH)], so[b])
            pltpu.async_copy(rbb.at[b], rb_hbm.at[pl.ds(base // 8, CH // 8)], so[b])
            pf = jnp.minimum(cc + 2, NCH - 1)
            issue_in(pf, b)
        return ()

    lax.fori_loop(0, NCH // 2, chunk, ())

    for b in range(2):
        g1, g2 = pend[b]
        g1.wait()
        g2.wait()
        pltpu.make_async_copy(shb.at[b], sh_hbm.at[pl.ds(0, CH)], so[b]).wait()
        pltpu.make_async_copy(rbb.at[b], rb_hbm.at[pl.ds(0, CH // 8)], so[b]).wait()

    for nc in range(NODES_W // 64):
        nb = wid * NODES_W + nc * 64
        pltpu.sync_copy(z_hbm.at[pl.ds(nb, 64)], zidx)
        pltpu.async_copy(emb_hbm.at[zidx], embr, sa0).wait()
        pltpu.sync_copy(embr, h0_hbm.at[pl.ds(nb, 64)])


def _geom_consts():
    ii = [[3, 0, 1, 2, 0, 1, 2, 0, 0] + [3] * 7,
          [3, 3, 3, 3, 1, 2, 2, 2, 0] + [3] * 7,
          [3] * 6 + [3, 3, 1] + [3] * 7,
          [3] * 6 + [3, 3, 1] + [3] * 7]
    ci = jnp.array(ii, jnp.int32)
    k1 = [1.0, _SQ3, _SQ3, _SQ3, _SQ15, _SQ15, 3.0 * _SQ5H, _SQ15, _SQ15H] + [0.0] * 7
    k2 = [0.0] * 6 + [-_SQ5H, 0.0, -_SQ15H] + [0.0] * 7
    oneh3 = [0.0] * 3 + [1.0] + [0.0] * 12
    steps = [r * _RBF_W for r in range(N_RBF)] + [1e6] * 6
    cf = jnp.array([k1, k2, oneh3, steps], jnp.float32)
    return ci, cf


NAGG = N_NODES


@functools.partial(
    pl.kernel,
    out_type=jax.ShapeDtypeStruct((NC, NAGG, MSGW), jnp.float32),
    mesh=_mesh,
    scratch_types=[
        pltpu.VMEM((NCH, CH), jnp.int32),
        pltpu.VMEM((NCH, CH), jnp.int32),
        pltpu.VMEM((2, CH, 16), jnp.float32),
        pltpu.VMEM((2, CH // 8, 128), jnp.float32),
        pltpu.VMEM((2, CH, 16), jnp.float32),
        pltpu.VMEM((2, CH, MSGW), jnp.float32),
        pltpu.VMEM_SHARED((NAGG, MSGW), jnp.float32),
        pltpu.SemaphoreType.DMA,
        pltpu.SemaphoreType.DMA,
        pltpu.SemaphoreType.DMA,
        pltpu.SemaphoreType.DMA,
        pltpu.SemaphoreType.DMA,
        pltpu.SemaphoreType.DMA,
    ],
    compiler_params=pltpu.CompilerParams(use_tc_tiling_on_sc=False),
)
def _sc_edge(src2_hbm, dst2_hbm, hw_hbm, rw_hbm, sh_hbm, out_hbm,
             sidx, didx, hwb, rwb, shb, msgb, agg_sh,
             sg0, sg1, sl0, sl1, ss0, ss1):
    cid = lax.axis_index("c")
    sid = lax.axis_index("s")
    zero16 = jnp.zeros((LANES,), jnp.float32)
    sg = (sg0, sg1)
    sl = (sl0, sl1)
    ss = (ss0, ss1)

    def zmsg(r, _):
        for cc in range(SH_DIM):
            msgb[0, r, pl.ds(cc * 16, 16)] = zero16
        return ()
    lax.fori_loop(0, CH, zmsg, ())
    for r in range(7):
        pltpu.sync_copy(msgb.at[0], agg_sh.at[pl.ds(sid * 625 + r * CH, CH)])
    pltpu.sync_copy(msgb.at[0].at[pl.ds(0, 65)],
                    agg_sh.at[pl.ds(sid * 625 + 7 * CH, 65)])
    plsc.subcore_barrier()

    wrow0 = (cid * NS + sid) * NCH
    pltpu.sync_copy(src2_hbm.at[pl.ds(wrow0, NCH)], sidx)
    pltpu.sync_copy(dst2_hbm.at[pl.ds(wrow0, NCH)], didx)

    def issue_in(ci, b):
        g = pltpu.async_copy(hw_hbm.at[sidx.at[ci]], hwb.at[b], sg[b])
        l1 = pltpu.async_copy(rw_hbm.at[pl.ds((wrow0 + ci) * (CH // 8), CH // 8)], rwb.at[b], sl[b])
        l2 = pltpu.async_copy(sh_hbm.at[pl.ds((wrow0 + ci) * CH, CH)], shb.at[b], sl[b])
        return g, l1, l2

    pend = [issue_in(0, 0), issue_in(1, 1)]

    def outer(go, _):
        for b in range(2):
            ci = go * 2 + b
            pltpu.make_async_copy(sh_hbm.at[pl.ds(0, CH)], hwb.at[b], sg[b]).wait()
            pltpu.make_async_copy(rw_hbm.at[pl.ds(0, CH // 8)], rwb.at[b], sl[b]).wait()
            pltpu.make_async_copy(sh_hbm.at[pl.ds(0, CH)], shb.at[b], sl[b]).wait()

            @pl.when(ci >= 2)
            def _():
                pltpu.make_async_copy(msgb.at[b], agg_sh.at[didx.at[0]],
                                      ss[b]).wait()

            def group(gg, _):
                for j in range(LANES):
                    i = gg * LANES + j
                    m = hwb[b, i] * rwb[b, 2 * gg + j // 8, pl.ds((j % 8) * 16, 16)]
                    shr = shb[b, i]
                    for k in range(SH_DIM):
                        sk = shr.at[jnp.full((LANES,), k, jnp.int32)].get(
                            mode='promise_in_bounds')
                        msgb[b, i, pl.ds(k * 16, 16)] = m * sk
                return ()
            lax.fori_loop(0, CH // LANES, group, ())
            pltpu.async_copy(msgb.at[b], agg_sh.at[didx.at[ci]], ss[b], add=True)
            pf = jnp.minimum(ci + 2, NCH - 1)
            issue_in(pf, b)
        return ()

    lax.fori_loop(0, NCH // 2, outer, ())

    for b in range(2):
        g, l1, l2 = pend[b]
        g.wait()
        l1.wait()
        l2.wait()
        pltpu.make_async_copy(msgb.at[b], agg_sh.at[didx.at[0]], ss[b]).wait()
    plsc.subcore_barrier()
    pltpu.sync_copy(agg_sh.at[pl.ds(sid * 625, 625)],
                    out_hbm.at[cid, pl.ds(sid * 625, 625)])


@functools.partial(
    pl.kernel,
    out_type=jax.ShapeDtypeStruct((N_GRAPHS, HID), jnp.float32),
    mesh=_mesh,
    scratch_types=[
        pltpu.VMEM((8,), jnp.int32),
        pltpu.VMEM((8, HID), jnp.float32),
        pltpu.SemaphoreType.DMA,
    ],
    compiler_params=pltpu.CompilerParams(use_tc_tiling_on_sc=False),
)
def _sc_gather_rows(h_hbm, idx_hbm, out_hbm, idxb, rows, sem0):
    wid = lax.axis_index("c") * NS + lax.axis_index("s")
    pltpu.sync_copy(idx_hbm.at[pl.ds(wid * 8, 8)], idxb)
    pltpu.async_copy(h_hbm.at[idxb], rows, sem0).wait()
    pltpu.sync_copy(rows, out_hbm.at[pl.ds(wid * 8, 8)])


def _silu(x):
    return x / (1.0 + jnp.exp(-x))


def _tc_radial_body(rb_ref, w1_ref, w2_ref, out_ref):
    e = pl.program_id(0)
    rb = rb_ref[...]
    rows = lax.broadcasted_iota(jnp.int32, (256, 1), 0) + e * 256
    msk = (rows < N_EDGES * 16 // 128).astype(jnp.float32)
    pieces = [[] for _ in range(N_LAYERS)]
    for g in range(8):
        t = _silu(jnp.dot(rb[:, g * 16:(g + 1) * 16], w1_ref[...],
                          preferred_element_type=jnp.float32))
        for l in range(N_LAYERS):
            pieces[l].append(jnp.dot(t[:, l * 32:(l + 1) * 32], w2_ref[l],
                                     preferred_element_type=jnp.float32))
    for l in range(N_LAYERS):
        out_ref[l] = jnp.concatenate(pieces[l], axis=1) * msk


def _tc_radial(rb, rw1cat, rw2s):
    return pl.pallas_call(
        _tc_radial_body,
        grid=(E_PAD // 8 // 256,),
        in_specs=[
            pl.BlockSpec((256, 128), lambda i: (i, 0)),
            pl.BlockSpec((16, 128), lambda i: (0, 0)),
            pl.BlockSpec((N_LAYERS, 32, 16), lambda i: (0, 0, 0)),
        ],
        out_specs=pl.BlockSpec((N_LAYERS, 256, 128), lambda i: (0, i, 0)),
        out_shape=jax.ShapeDtypeStruct((N_LAYERS, E_PAD // 8, 128), jnp.float32),
    )(rb, rw1cat, rw2s)


def _tc_hw_body(h_ref, w_ref, out_ref):
    out_ref[...] = jnp.dot(h_ref[...], w_ref[...],
                           preferred_element_type=jnp.float32)


def _tc_hw(h, w):
    n = h.shape[0]
    return pl.pallas_call(
        _tc_hw_body,
        grid=(n // 1000,),
        in_specs=[
            pl.BlockSpec((1000, h.shape[1]), lambda i: (i, 0)),
            pl.BlockSpec(w.shape, lambda i: (0, 0)),
        ],
        out_specs=pl.BlockSpec((1000, 16), lambda i: (i, 0)),
        out_shape=jax.ShapeDtypeStruct((n, 16), jnp.float32),
    )(h, w)


def _tc_node_body(agg_ref, h_ref, wout_ref, wsc_ref, wmsg_ref, h_o, hw_o):
    a = agg_ref[0] + agg_ref[1]
    hn = (jnp.dot(a, wout_ref[...], preferred_element_type=jnp.float32)
          + jnp.dot(h_ref[...], wsc_ref[...], preferred_element_type=jnp.float32))
    h_o[...] = hn
    hw_o[...] = jnp.dot(hn, wmsg_ref[...], preferred_element_type=jnp.float32)


def _tc_node(agg2, h, wout, wsc, wmsg):
    d_in = h.shape[1]
    return pl.pallas_call(
        _tc_node_body,
        grid=(N_NODES // 1000,),
        in_specs=[
            pl.BlockSpec((NC, 1000, MSGW), lambda i: (0, i, 0)),
            pl.BlockSpec((1000, d_in), lambda i: (i, 0)),
            pl.BlockSpec((MSGW, HID), lambda i: (0, 0)),
            pl.BlockSpec((d_in, HID), lambda i: (0, 0)),
            pl.BlockSpec((HID, 16), lambda i: (0, 0)),
        ],
        out_specs=[
            pl.BlockSpec((1000, HID), lambda i: (i, 0)),
            pl.BlockSpec((1000, 16), lambda i: (i, 0)),
        ],
        out_shape=[
            jax.ShapeDtypeStruct((N_NODES, HID), jnp.float32),
            jax.ShapeDtypeStruct((N_NODES, 16), jnp.float32),
        ],
    )(agg2, h, wout, wsc, wmsg)


def _tc_readout_body(h_ref, ha_ref, ga_ref, batch_ref,
                     wq_ref, wk_ref, wv_ref,
                     w1s_ref, w1c_ref, w1v_ref, w1t_ref, b1_ref,
                     w2_ref, b2_ref, s3_ref, s5_ref, out_ref):
    scal = h_ref[:, :MUL0]
    k = jnp.dot(scal, wk_ref[...], preferred_element_type=jnp.float32)
    v = jnp.dot(scal, wv_ref[...], preferred_element_type=jnp.float32)
    sa = ha_ref[:, :MUL0]
    q = jnp.dot(sa, wq_ref[...], preferred_element_type=jnp.float32)
    scores = lax.dot_general(q, k, (((1,), (1,)), ((), ())),
                             preferred_element_type=jnp.float32)
    scores = scores * (1.0 / (MUL0 ** 0.5))
    valid = ga_ref[...] == batch_ref[...]
    scores = jnp.where(valid, scores, -1e9)
    mx = jnp.max(scores, axis=1, keepdims=True)
    e = jnp.exp(scores - mx)
    attn = e / jnp.sum(e, axis=1, keepdims=True)
    c = jnp.dot(attn, v, preferred_element_type=jnp.float32)
    vsq = ha_ref[:, MUL0:MUL0 + MUL1 * 3]
    nv = jnp.dot(vsq * vsq, s3_ref[...], preferred_element_type=jnp.float32)
    tsq = ha_ref[:, MUL0 + MUL1 * 3:HID]
    nt = jnp.dot(tsq * tsq, s5_ref[...], preferred_element_type=jnp.float32)
    zr = (jnp.dot(sa, w1s_ref[...], preferred_element_type=jnp.float32)
          + jnp.dot(c, w1c_ref[...], preferred_element_type=jnp.float32)
          + jnp.dot(nv, w1v_ref[...], preferred_element_type=jnp.float32)
          + jnp.dot(nt, w1t_ref[...], preferred_element_type=jnp.float32)
          + b1_ref[...])
    hdn = _silu(zr)
    out_ref[...] = jnp.dot(hdn, w2_ref[...],
                           preferred_element_type=jnp.float32) + b2_ref[...]


def _tc_readout(h, ha, ga2, batch2, wq, wk, wv, w1s, w1c, w1v, w1t, b1, w2, b2,
                s3, s5):
    return pl.pallas_call(
        _tc_readout_body,
        out_shape=jax.ShapeDtypeStruct((N_GRAPHS, NUM_BASIS), jnp.float32),
    )(h, ha, ga2, batch2, wq, wk, wv, w1s, w1c, w1v, w1t, b1, w2, b2, s3, s5)


def kernel(z, pos, edge_index, batch, absorber_mask, params):
    f32 = jnp.float32
    z_pad = jnp.pad(z.astype(jnp.int32), (0, N_PAD - N_NODES))
    pos16 = jnp.zeros((N_PAD, 16), f32).at[:N_NODES, :3].set(pos)
    src = jnp.pad(edge_index[0].astype(jnp.int32), (0, E_PAD - N_EDGES))
    dst = jnp.pad(edge_index[1].astype(jnp.int32), (0, E_PAD - N_EDGES))
    abs_idx = jnp.nonzero(absorber_mask, size=N_GRAPHS)[0].astype(jnp.int32)
    g_a = batch[abs_idx].astype(jnp.int32)

    layers = params['layers']
    rw1cat = jnp.concatenate(
        [jnp.pad(lp['rw1'], ((0, 16 - N_RBF), (0, 0))) for lp in layers], axis=1)
    rw2s = jnp.stack([lp['rw2'] for lp in layers])
    perm = (jnp.arange(MSGW) % 16) * SH_DIM + (jnp.arange(MSGW) // 16)
    wouts = [lp['w_out'][perm] * 0.25 for lp in layers]
    s3 = (jnp.arange(MUL1 * 3)[:, None] // 3 == jnp.arange(MUL1)[None, :]).astype(f32)
    s5 = (jnp.arange(MUL2 * 5)[:, None] // 5 == jnp.arange(MUL2)[None, :]).astype(f32)
    w1 = params['w1']
    w1s, w1c = w1[:MUL0], w1[MUL0:2 * MUL0]
    w1v, w1t = w1[2 * MUL0:2 * MUL0 + MUL1], w1[2 * MUL0 + MUL1:]

    ci_const, cf_const = _geom_consts()
    src2 = src.reshape(E_PAD // CH, CH)
    dst2 = dst.reshape(E_PAD // CH, CH)
    sh, rb, h0 = _sc_geom(pos16, src2, dst2, z_pad, params['emb'], ci_const, cf_const)
    rw_all = _tc_radial(rb, rw1cat, rw2s)
    h = h0[:N_NODES]
    hw = _tc_hw(h, layers[0]['w_msg'])
    for l in range(N_LAYERS):
        agg2 = _sc_edge(src2, dst2, hw, rw_all[l], sh)
        wmsg_next = (layers[l + 1]['w_msg'] if l + 1 < N_LAYERS
                     else jnp.zeros((HID, 16), f32))
        h, hw = _tc_node(agg2, h, wouts[l], layers[l]['w_sc'], wmsg_next)

    ha = _sc_gather_rows(h, abs_idx)
    return _tc_readout(h, ha, g_a[:, None], batch.astype(jnp.int32)[None, :],
                       params['wq'], params['wk'], params['wv'],
                       w1s, w1c, w1v, w1t, params['b1'][None, :],
                       params['w2'], params['b2'][None, :], s3, s5)


# speedup vs baseline: 98.4483x; 1.1565x over previous
import functools

import jax
import jax.numpy as jnp
from jax import lax
from jax.experimental import pallas as pl
from jax.experimental.pallas import tpu as pltpu
from jax.experimental.pallas import tpu_sc as plsc

N_NODES = 10000
N_EDGES = 160000
N_GRAPHS = 256
MUL0, MUL1, MUL2 = 64, 32, 16
HID = 240
MUL_MSG = 16
SH_DIM = 9
N_RBF = 10
R_MAX = 5.0
NUM_BASIS = 128
N_LAYERS = 4

NC, NS, LANES = 2, 16, 16
NW = NC * NS
N_PAD = 10240
E_PAD = 163840
EW = E_PAD // NW
CH = 80
NCH = EW // CH
NODES_W = N_PAD // NW
MSGW = MUL_MSG * SH_DIM

_SQ3 = 3.0 ** 0.5
_SQ15 = 15.0 ** 0.5
_SQ5H = (5.0 ** 0.5) / 2.0
_SQ15H = _SQ15 / 2.0
_RBF_W = R_MAX / (N_RBF - 1)
_RBF_C = -1.0 / (2.0 * _RBF_W * _RBF_W)

_mesh = plsc.VectorSubcoreMesh(core_axis_name="c", subcore_axis_name="s",
                               num_cores=NC, num_subcores=NS)


def _rsqrt_nr(l2):
    i = lax.bitcast_convert_type(l2, jnp.int32)
    i = jnp.int32(0x5F3759DF) - (i >> 1)
    y = lax.bitcast_convert_type(i, jnp.float32)
    for _ in range(3):
        y = y * (1.5 - 0.5 * l2 * y * y)
    return y


@functools.partial(
    pl.kernel,
    out_type=(jax.ShapeDtypeStruct((E_PAD, 16), jnp.float32),
              jax.ShapeDtypeStruct((E_PAD // 8, 128), jnp.float32),
              jax.ShapeDtypeStruct((N_PAD, MUL0), jnp.float32)),
    mesh=_mesh,
    scratch_types=[
        pltpu.VMEM((4, 16), jnp.int32),
        pltpu.VMEM((4, 16), jnp.float32),
        pltpu.VMEM((NCH, CH), jnp.int32),
        pltpu.VMEM((NCH, CH), jnp.int32),
        pltpu.VMEM((2, CH, 16), jnp.float32),
        pltpu.VMEM((2, CH, 16), jnp.float32),
        pltpu.VMEM((2, CH, 16), jnp.float32),
        pltpu.VMEM((2, CH // 8, 128), jnp.float32),
        pltpu.VMEM((64,), jnp.int32),
        pltpu.VMEM((64, MUL0), jnp.float32),
        pltpu.SemaphoreType.DMA,
        pltpu.SemaphoreType.DMA,
        pltpu.SemaphoreType.DMA,
        pltpu.SemaphoreType.DMA,
        pltpu.SemaphoreType.DMA,
        pltpu.SemaphoreType.DMA,
    ],
    compiler_params=pltpu.CompilerParams(use_tc_tiling_on_sc=False),
)
def _sc_geom(pos_hbm, src2_hbm, dst2_hbm, z_hbm, emb_hbm, ci_hbm, cf_hbm,
             sh_hbm, rb_hbm, h0_hbm,
             cib, cfb, sidx, didx, ps, pd, shb, rbb, zidx, embr,
             sa0, sa1, sb0, sb1, so0, so1):
    wid = lax.axis_index("c") * NS + lax.axis_index("s")
    pltpu.sync_copy(ci_hbm, cib)
    pltpu.sync_copy(cf_hbm, cfb)
    ia, ib, ic, idd = cib[0], cib[1], cib[2], cib[3]
    k1, k2, oneh3, steps = cfb[0], cfb[1], cfb[2], cfb[3]
    l0 = jnp.zeros((LANES,), jnp.int32)
    l1v = jnp.full((LANES,), 1, jnp.int32)
    l2i = jnp.full((LANES,), 2, jnp.int32)
    sa = (sa0, sa1)
    sb = (sb0, sb1)
    so = (so0, so1)
    wrow0 = wid * NCH

    def _g(x, idx):
        return x.at[idx].get(mode='promise_in_bounds')

    pltpu.sync_copy(src2_hbm.at[pl.ds(wrow0, NCH)], sidx)
    pltpu.sync_copy(dst2_hbm.at[pl.ds(wrow0, NCH)], didx)

    def issue_in(ci, b):
        g1 = pltpu.async_copy(pos_hbm.at[sidx.at[ci]], ps.at[b], sa[b])
        g2 = pltpu.async_copy(pos_hbm.at[didx.at[ci]], pd.at[b], sb[b])
        return g1, g2

    pend = [issue_in(0, 0), issue_in(1, 1)]

    def chunk(ci, _):
        for b in range(2):
            cc = ci * 2 + b
            base = (wrow0 + cc) * CH
            pltpu.make_async_copy(sh_hbm.at[pl.ds(0, CH)], ps.at[b], sa[b]).wait()
            pltpu.make_async_copy(sh_hbm.at[pl.ds(0, CH)], pd.at[b], sb[b]).wait()

            @pl.when(cc >= 2)
            def _():
                pltpu.make_async_copy(shb.at[b], sh_hbm.at[pl.ds(0, CH)], so[b]).wait()
                pltpu.make_async_copy(rbb.at[b], rb_hbm.at[pl.ds(0, CH // 8)], so[b]).wait()

            def edge(g, _):
                for j in range(LANES):
                    i = g * LANES + j
                    dv = pd[b, i] - ps[b, i]
                    sq = dv * dv
                    l2 = _g(sq, l0) + _g(sq, l1v) + _g(sq, l2i) + 1e-12
                    rs = _rsqrt_nr(l2)
                    t = dv * rs + oneh3
                    shb[b, i] = (k1 * _g(t, ia) * _g(t, ib)
                                 + k2 * _g(t, ic) * _g(t, idd))
                    dd = l2 * rs - steps
                    rbb[b, 2 * g + j // 8, pl.ds((j % 8) * 16, 16)] = (
                        jnp.exp(dd * dd * _RBF_C))
                return ()
            lax.fori_loop(0, CH // LANES, edge, ())
            pltpu.async_copy(shb.at[b], sh_hbm.at[pl.ds(base, CH)], so[b])
            pltpu.async_copy(rbb.at[b], rb_hbm.at[pl.ds(base // 8, CH // 8)], so[b])
            pf = jnp.minimum(cc + 2, NCH - 1)
            issue_in(pf, b)
        return ()

    lax.fori_loop(0, NCH // 2, chunk, ())

    for b in range(2):
        g1, g2 = pend[b]
        g1.wait()
        g2.wait()
        pltpu.make_async_copy(shb.at[b], sh_hbm.at[pl.ds(0, CH)], so[b]).wait()
        pltpu.make_async_copy(rbb.at[b], rb_hbm.at[pl.ds(0, CH // 8)], so[b]).wait()

    for nc in range(NODES_W // 64):
        nb = wid * NODES_W + nc * 64
        pltpu.sync_copy(z_hbm.at[pl.ds(nb, 64)], zidx)
        pltpu.async_copy(emb_hbm.at[zidx], embr, sa0).wait()
        pltpu.sync_copy(embr, h0_hbm.at[pl.ds(nb, 64)])


def _geom_consts():
    ii = [[3, 0, 1, 2, 0, 1, 2, 0, 0] + [3] * 7,
          [3, 3, 3, 3, 1, 2, 2, 2, 0] + [3] * 7,
          [3] * 6 + [3, 3, 1] + [3] * 7,
          [3] * 6 + [3, 3, 1] + [3] * 7]
    ci = jnp.array(ii, jnp.int32)
    k1 = [1.0, _SQ3, _SQ3, _SQ3, _SQ15, _SQ15, 3.0 * _SQ5H, _SQ15, _SQ15H] + [0.0] * 7
    k2 = [0.0] * 6 + [-_SQ5H, 0.0, -_SQ15H] + [0.0] * 7
    oneh3 = [0.0] * 3 + [1.0] + [0.0] * 12
    steps = [r * _RBF_W for r in range(N_RBF)] + [1e6] * 6
    cf = jnp.array([k1, k2, oneh3, steps], jnp.float32)
    return ci, cf


NAGG = N_NODES


@functools.partial(
    pl.kernel,
    out_type=jax.ShapeDtypeStruct((NC, NAGG, MSGW), jnp.float32),
    mesh=_mesh,
    scratch_types=[
        pltpu.VMEM((NCH, CH), jnp.int32),
        pltpu.VMEM((NCH, CH), jnp.int32),
        pltpu.VMEM((2, CH, 16), jnp.float32),
        pltpu.VMEM((2, CH // 8, 128), jnp.float32),
        pltpu.VMEM((2, CH, 16), jnp.float32),
        pltpu.VMEM((2, CH, MSGW), jnp.float32),
        pltpu.VMEM_SHARED((NAGG, MSGW), jnp.float32),
        pltpu.SemaphoreType.DMA,
        pltpu.SemaphoreType.DMA,
        pltpu.SemaphoreType.DMA,
        pltpu.SemaphoreType.DMA,
        pltpu.SemaphoreType.DMA,
        pltpu.SemaphoreType.DMA,
    ],
    compiler_params=pltpu.CompilerParams(use_tc_tiling_on_sc=False),
)
def _sc_edge(src2_hbm, dst2_hbm, hw_hbm, rw_hbm, sh_hbm, out_hbm,
             sidx, didx, hwb, rwb, shb, msgb, agg_sh,
             sg0, sg1, sl0, sl1, ss0, ss1):
    cid = lax.axis_index("c")
    sid = lax.axis_index("s")
    zero16 = jnp.zeros((LANES,), jnp.float32)
    sg = (sg0, sg1)
    sl = (sl0, sl1)
    ss = (ss0, ss1)

    def zmsg(r, _):
        for cc in range(SH_DIM):
            msgb[0, r, pl.ds(cc * 16, 16)] = zero16
        return ()
    lax.fori_loop(0, CH, zmsg, ())
    for r in range(7):
        pltpu.sync_copy(msgb.at[0], agg_sh.at[pl.ds(sid * 625 + r * CH, CH)])
    pltpu.sync_copy(msgb.at[0].at[pl.ds(0, 65)],
                    agg_sh.at[pl.ds(sid * 625 + 7 * CH, 65)])
    plsc.subcore_barrier()

    wrow0 = (cid * NS + sid) * NCH
    pltpu.sync_copy(src2_hbm.at[pl.ds(wrow0, NCH)], sidx)
    pltpu.sync_copy(dst2_hbm.at[pl.ds(wrow0, NCH)], didx)

    def issue_in(ci, b):
        g = pltpu.async_copy(hw_hbm.at[sidx.at[ci]], hwb.at[b], sg[b])
        l1 = pltpu.async_copy(rw_hbm.at[pl.ds((wrow0 + ci) * (CH // 8), CH // 8)], rwb.at[b], sl[b])
        l2 = pltpu.async_copy(sh_hbm.at[pl.ds((wrow0 + ci) * CH, CH)], shb.at[b], sl[b])
        return g, l1, l2

    pend = [issue_in(0, 0), issue_in(1, 1)]

    def outer(go, _):
        for b in range(2):
            ci = go * 2 + b
            pltpu.make_async_copy(sh_hbm.at[pl.ds(0, CH)], hwb.at[b], sg[b]).wait()
            pltpu.make_async_copy(rw_hbm.at[pl.ds(0, CH // 8)], rwb.at[b], sl[b]).wait()
            pltpu.make_async_copy(sh_hbm.at[pl.ds(0, CH)], shb.at[b], sl[b]).wait()

            @pl.when(ci >= 2)
            def _():
                pltpu.make_async_copy(msgb.at[b], agg_sh.at[didx.at[0]],
                                      ss[b]).wait()

            def group(gg, _):
                for j in range(LANES):
                    i = gg * LANES + j
                    m = hwb[b, i] * rwb[b, 2 * gg + j // 8, pl.ds((j % 8) * 16, 16)]
                    shr = shb[b, i]
                    for k in range(SH_DIM):
                        sk = shr.at[jnp.full((LANES,), k, jnp.int32)].get(
                            mode='promise_in_bounds')
                        msgb[b, i, pl.ds(k * 16, 16)] = m * sk
                return ()
            lax.fori_loop(0, CH // LANES, group, ())
            pltpu.async_copy(msgb.at[b], agg_sh.at[didx.at[ci]], ss[b], add=True)
            pf = jnp.minimum(ci + 2, NCH - 1)
            issue_in(pf, b)
        return ()

    lax.fori_loop(0, NCH // 2, outer, ())

    for b in range(2):
        g, l1, l2 = pend[b]
        g.wait()
        l1.wait()
        l2.wait()
        pltpu.make_async_copy(msgb.at[b], agg_sh.at[didx.at[0]], ss[b]).wait()
    plsc.subcore_barrier()
    pltpu.sync_copy(agg_sh.at[pl.ds(sid * 625, 625)],
                    out_hbm.at[cid, pl.ds(sid * 625, 625)])


@functools.partial(
    pl.kernel,
    out_type=jax.ShapeDtypeStruct((N_GRAPHS, HID), jnp.float32),
    mesh=_mesh,
    scratch_types=[
        pltpu.VMEM((8,), jnp.int32),
        pltpu.VMEM((8, HID), jnp.float32),
        pltpu.SemaphoreType.DMA,
    ],
    compiler_params=pltpu.CompilerParams(use_tc_tiling_on_sc=False),
)
def _sc_gather_rows(h_hbm, idx_hbm, out_hbm, idxb, rows, sem0):
    wid = lax.axis_index("c") * NS + lax.axis_index("s")
    pltpu.sync_copy(idx_hbm.at[pl.ds(wid * 8, 8)], idxb)
    pltpu.async_copy(h_hbm.at[idxb], rows, sem0).wait()
    pltpu.sync_copy(rows, out_hbm.at[pl.ds(wid * 8, 8)])


def _silu(x):
    return x / (1.0 + jnp.exp(-x))


def _tc_radial_body(rb_ref, bd1_ref, bd2_ref, out_ref):
    e = pl.program_id(0)
    rb = rb_ref[...]
    rows = lax.broadcasted_iota(jnp.int32, (256, 1), 0) + e * 256
    msk = (rows < N_EDGES * 16 // 128).astype(jnp.float32)
    t = _silu(jnp.dot(rb, bd1_ref[...], preferred_element_type=jnp.float32))
    tb = t.astype(jnp.bfloat16)
    for l in range(N_LAYERS):
        out_ref[l] = jnp.dot(tb, bd2_ref[l],
                             preferred_element_type=jnp.float32) * msk


def _tc_radial(rb, bd1, bd2):
    return pl.pallas_call(
        _tc_radial_body,
        grid=(E_PAD // 8 // 256,),
        in_specs=[
            pl.BlockSpec((256, 128), lambda i: (i, 0)),
            pl.BlockSpec((128, 1024), lambda i: (0, 0)),
            pl.BlockSpec((N_LAYERS, 1024, 128), lambda i: (0, 0, 0)),
        ],
        out_specs=pl.BlockSpec((N_LAYERS, 256, 128), lambda i: (0, i, 0)),
        out_shape=jax.ShapeDtypeStruct((N_LAYERS, E_PAD // 8, 128), jnp.float32),
    )(rb, bd1, bd2)


def _tc_hw_body(h_ref, w_ref, out_ref):
    out_ref[...] = jnp.dot(h_ref[...], w_ref[...],
                           preferred_element_type=jnp.float32)


def _tc_hw(h, w):
    n = h.shape[0]
    return pl.pallas_call(
        _tc_hw_body,
        grid=(n // 1000,),
        in_specs=[
            pl.BlockSpec((1000, h.shape[1]), lambda i: (i, 0)),
            pl.BlockSpec(w.shape, lambda i: (0, 0)),
        ],
        out_specs=pl.BlockSpec((1000, 16), lambda i: (i, 0)),
        out_shape=jax.ShapeDtypeStruct((n, 16), jnp.float32),
    )(h, w)


def _tc_node_body(agg_ref, h_ref, wout_ref, wsc_ref, wmsg_ref, h_o, hw_o):
    a = agg_ref[0] + agg_ref[1]
    hn = (jnp.dot(a, wout_ref[...], preferred_element_type=jnp.float32)
          + jnp.dot(h_ref[...], wsc_ref[...], preferred_element_type=jnp.float32))
    h_o[...] = hn
    hw_o[...] = jnp.dot(hn, wmsg_ref[...], preferred_element_type=jnp.float32)


def _tc_node(agg2, h, wout, wsc, wmsg):
    d_in = h.shape[1]
    return pl.pallas_call(
        _tc_node_body,
        grid=(N_NODES // 1000,),
        in_specs=[
            pl.BlockSpec((NC, 1000, MSGW), lambda i: (0, i, 0)),
            pl.BlockSpec((1000, d_in), lambda i: (i, 0)),
            pl.BlockSpec((MSGW, HID), lambda i: (0, 0)),
            pl.BlockSpec((d_in, HID), lambda i: (0, 0)),
            pl.BlockSpec((HID, 16), lambda i: (0, 0)),
        ],
        out_specs=[
            pl.BlockSpec((1000, HID), lambda i: (i, 0)),
            pl.BlockSpec((1000, 16), lambda i: (i, 0)),
        ],
        out_shape=[
            jax.ShapeDtypeStruct((N_NODES, HID), jnp.float32),
            jax.ShapeDtypeStruct((N_NODES, 16), jnp.float32),
        ],
    )(agg2, h, wout, wsc, wmsg)


def _tc_readout_body(h_ref, ha_ref, ga_ref, batch_ref,
                     wq_ref, wk_ref, wv_ref,
                     w1s_ref, w1c_ref, w1v_ref, w1t_ref, b1_ref,
                     w2_ref, b2_ref, s3_ref, s5_ref, out_ref):
    scal = h_ref[:, :MUL0]
    k = jnp.dot(scal, wk_ref[...], preferred_element_type=jnp.float32)
    v = jnp.dot(scal, wv_ref[...], preferred_element_type=jnp.float32)
    sa = ha_ref[:, :MUL0]
    q = jnp.dot(sa, wq_ref[...], preferred_element_type=jnp.float32)
    scores = lax.dot_general(q, k, (((1,), (1,)), ((), ())),
                             preferred_element_type=jnp.float32)
    scores = scores * (1.0 / (MUL0 ** 0.5))
    valid = ga_ref[...] == batch_ref[...]
    scores = jnp.where(valid, scores, -1e9)
    mx = jnp.max(scores, axis=1, keepdims=True)
    e = jnp.exp(scores - mx)
    attn = e / jnp.sum(e, axis=1, keepdims=True)
    c = jnp.dot(attn, v, preferred_element_type=jnp.float32)
    vsq = ha_ref[:, MUL0:MUL0 + MUL1 * 3]
    nv = jnp.dot(vsq * vsq, s3_ref[...], preferred_element_type=jnp.float32)
    tsq = ha_ref[:, MUL0 + MUL1 * 3:HID]
    nt = jnp.dot(tsq * tsq, s5_ref[...], preferred_element_type=jnp.float32)
    zr = (jnp.dot(sa, w1s_ref[...], preferred_element_type=jnp.float32)
          + jnp.dot(c, w1c_ref[...], preferred_element_type=jnp.float32)
          + jnp.dot(nv, w1v_ref[...], preferred_element_type=jnp.float32)
          + jnp.dot(nt, w1t_ref[...], preferred_element_type=jnp.float32)
          + b1_ref[...])
    hdn = _silu(zr)
    out_ref[...] = jnp.dot(hdn, w2_ref[...],
                           preferred_element_type=jnp.float32) + b2_ref[...]


def _tc_readout(h, ha, ga2, batch2, wq, wk, wv, w1s, w1c, w1v, w1t, b1, w2, b2,
                s3, s5):
    return pl.pallas_call(
        _tc_readout_body,
        out_shape=jax.ShapeDtypeStruct((N_GRAPHS, NUM_BASIS), jnp.float32),
    )(h, ha, ga2, batch2, wq, wk, wv, w1s, w1c, w1v, w1t, b1, w2, b2, s3, s5)


def kernel(z, pos, edge_index, batch, absorber_mask, params):
    f32 = jnp.float32
    z_pad = jnp.pad(z.astype(jnp.int32), (0, N_PAD - N_NODES))
    pos16 = jnp.zeros((N_PAD, 16), f32).at[:N_NODES, :3].set(pos)
    src = jnp.pad(edge_index[0].astype(jnp.int32), (0, E_PAD - N_EDGES))
    dst = jnp.pad(edge_index[1].astype(jnp.int32), (0, E_PAD - N_EDGES))
    abs_idx = jnp.nonzero(absorber_mask, size=N_GRAPHS)[0].astype(jnp.int32)
    g_a = batch[abs_idx].astype(jnp.int32)

    layers = params['layers']
    rw1cat = jnp.concatenate(
        [jnp.pad(lp['rw1'], ((0, 16 - N_RBF), (0, 0))) for lp in layers], axis=1)
    bd1 = jnp.zeros((128, 1024), f32)
    for e in range(8):
        bd1 = bd1.at[e * 16:(e + 1) * 16, e * 128:(e + 1) * 128].set(rw1cat)
    bd2 = jnp.zeros((N_LAYERS, 1024, 128), f32)
    for l in range(N_LAYERS):
        for e in range(8):
            bd2 = bd2.at[l, e * 128 + l * 32:e * 128 + (l + 1) * 32,
                         e * 16:(e + 1) * 16].set(layers[l]['rw2'])
    bd2 = bd2.astype(jnp.bfloat16)
    perm = (jnp.arange(MSGW) % 16) * SH_DIM + (jnp.arange(MSGW) // 16)
    wouts = [lp['w_out'][perm] * 0.25 for lp in layers]
    s3 = (jnp.arange(MUL1 * 3)[:, None] // 3 == jnp.arange(MUL1)[None, :]).astype(f32)
    s5 = (jnp.arange(MUL2 * 5)[:, None] // 5 == jnp.arange(MUL2)[None, :]).astype(f32)
    w1 = params['w1']
    w1s, w1c = w1[:MUL0], w1[MUL0:2 * MUL0]
    w1v, w1t = w1[2 * MUL0:2 * MUL0 + MUL1], w1[2 * MUL0 + MUL1:]

    ci_const, cf_const = _geom_consts()
    src2 = src.reshape(E_PAD // CH, CH)
    dst2 = dst.reshape(E_PAD // CH, CH)
    sh, rb, h0 = _sc_geom(pos16, src2, dst2, z_pad, params['emb'], ci_const, cf_const)
    rw_all = _tc_radial(rb, bd1, bd2)
    h = h0[:N_NODES]
    hw = _tc_hw(h, layers[0]['w_msg'])
    for l in range(N_LAYERS):
        agg2 = _sc_edge(src2, dst2, hw, rw_all[l], sh)
        wmsg_next = (layers[l + 1]['w_msg'] if l + 1 < N_LAYERS
                     else jnp.zeros((HID, 16), f32))
        h, hw = _tc_node(agg2, h, wouts[l], layers[l]['w_sc'], wmsg_next)

    ha = _sc_gather_rows(h, abs_idx)
    return _tc_readout(h, ha, g_a[:, None], batch.astype(jnp.int32)[None, :],
                       params['wq'], params['wk'], params['wv'],
                       w1s, w1c, w1v, w1t, params['b1'][None, :],
                       params['w2'], params['b2'][None, :], s3, s5)


# speedup vs baseline: 99.5930x; 1.0116x over previous
import functools

import jax
import jax.numpy as jnp
from jax import lax
from jax.experimental import pallas as pl
from jax.experimental.pallas import tpu as pltpu
from jax.experimental.pallas import tpu_sc as plsc

N_NODES = 10000
N_EDGES = 160000
N_GRAPHS = 256
MUL0, MUL1, MUL2 = 64, 32, 16
HID = 240
MUL_MSG = 16
SH_DIM = 9
N_RBF = 10
R_MAX = 5.0
NUM_BASIS = 128
N_LAYERS = 4

NC, NS, LANES = 2, 16, 16
NW = NC * NS
N_PAD = 10240
E_PAD = 163840
EW = E_PAD // NW
CH = 80
NCH = EW // CH
NODES_W = N_PAD // NW
MSGW = MUL_MSG * SH_DIM

_SQ3 = 3.0 ** 0.5
_SQ15 = 15.0 ** 0.5
_SQ5H = (5.0 ** 0.5) / 2.0
_SQ15H = _SQ15 / 2.0
_RBF_W = R_MAX / (N_RBF - 1)
_RBF_C = -1.0 / (2.0 * _RBF_W * _RBF_W)

_mesh = plsc.VectorSubcoreMesh(core_axis_name="c", subcore_axis_name="s",
                               num_cores=NC, num_subcores=NS)


def _rsqrt_nr(l2):
    i = lax.bitcast_convert_type(l2, jnp.int32)
    i = jnp.int32(0x5F3759DF) - (i >> 1)
    y = lax.bitcast_convert_type(i, jnp.float32)
    for _ in range(3):
        y = y * (1.5 - 0.5 * l2 * y * y)
    return y


@functools.partial(
    pl.kernel,
    out_type=(jax.ShapeDtypeStruct((E_PAD, 16), jnp.float32),
              jax.ShapeDtypeStruct((E_PAD // 8, 128), jnp.float32),
              jax.ShapeDtypeStruct((N_PAD, MUL0), jnp.float32)),
    mesh=_mesh,
    scratch_types=[
        pltpu.VMEM((4, 16), jnp.int32),
        pltpu.VMEM((4, 16), jnp.float32),
        pltpu.VMEM((NCH, CH), jnp.int32),
        pltpu.VMEM((NCH, CH), jnp.int32),
        pltpu.VMEM((2, CH, 16), jnp.float32),
        pltpu.VMEM((2, CH, 16), jnp.float32),
        pltpu.VMEM((2, CH, 16), jnp.float32),
        pltpu.VMEM((2, CH // 8, 128), jnp.float32),
        pltpu.VMEM((64,), jnp.int32),
        pltpu.VMEM((64, MUL0), jnp.float32),
        pltpu.SemaphoreType.DMA,
        pltpu.SemaphoreType.DMA,
        pltpu.SemaphoreType.DMA,
        pltpu.SemaphoreType.DMA,
        pltpu.SemaphoreType.DMA,
        pltpu.SemaphoreType.DMA,
    ],
    compiler_params=pltpu.CompilerParams(use_tc_tiling_on_sc=False),
)
def _sc_geom(pos_hbm, src2_hbm, dst2_hbm, z_hbm, emb_hbm, ci_hbm, cf_hbm,
             sh_hbm, rb_hbm, h0_hbm,
             cib, cfb, sidx, didx, ps, pd, shb, rbb, zidx, embr,
             sa0, sa1, sb0, sb1, so0, so1):
    wid = lax.axis_index("c") * NS + lax.axis_index("s")
    pltpu.sync_copy(ci_hbm, cib)
    pltpu.sync_copy(cf_hbm, cfb)
    ia, ib, ic, idd = cib[0], cib[1], cib[2], cib[3]
    k1, k2, oneh3, steps = cfb[0], cfb[1], cfb[2], cfb[3]
    l0 = jnp.zeros((LANES,), jnp.int32)
    l1v = jnp.full((LANES,), 1, jnp.int32)
    l2i = jnp.full((LANES,), 2, jnp.int32)
    sa = (sa0, sa1)
    sb = (sb0, sb1)
    so = (so0, so1)
    wrow0 = wid * NCH

    def _g(x, idx):
        return x.at[idx].get(mode='promise_in_bounds')

    pltpu.sync_copy(src2_hbm.at[pl.ds(wrow0, NCH)], sidx)
    pltpu.sync_copy(dst2_hbm.at[pl.ds(wrow0, NCH)], didx)

    def issue_in(ci, b):
        g1 = pltpu.async_copy(pos_hbm.at[sidx.at[ci]], ps.at[b], sa[b])
        g2 = pltpu.async_copy(pos_hbm.at[didx.at[ci]], pd.at[b], sb[b])
        return g1, g2

    pend = [issue_in(0, 0), issue_in(1, 1)]

    def chunk(ci, _):
        for b in range(2):
            cc = ci * 2 + b
            base = (wrow0 + cc) * CH
            pltpu.make_async_copy(sh_hbm.at[pl.ds(0, CH)], ps.at[b], sa[b]).wait()
            pltpu.make_async_copy(sh_hbm.at[pl.ds(0, CH)], pd.at[b], sb[b]).wait()

            @pl.when(cc >= 2)
            def _():
                pltpu.make_async_copy(shb.at[b], sh_hbm.at[pl.ds(0, CH)], so[b]).wait()
                pltpu.make_async_copy(rbb.at[b], rb_hbm.at[pl.ds(0, CH // 8)], so[b]).wait()

            def edge(g, _):
                for j in range(LANES):
                    i = g * LANES + j
                    dv = pd[b, i] - ps[b, i]
                    sq = dv * dv
                    l2 = _g(sq, l0) + _g(sq, l1v) + _g(sq, l2i) + 1e-12
                    rs = _rsqrt_nr(l2)
                    t = dv * rs + oneh3
                    shb[b, i] = (k1 * _g(t, ia) * _g(t, ib)
                                 + k2 * _g(t, ic) * _g(t, idd))
                    dd = l2 * rs - steps
                    rbb[b, 2 * g + j // 8, pl.ds((j % 8) * 16, 16)] = (
                        jnp.exp(dd * dd * _RBF_C))
                return ()
            lax.fori_loop(0, CH // LANES, edge, ())
            pltpu.async_copy(shb.at[b], sh_hbm.at[pl.ds(base, CH)], so[b])
            pltpu.async_copy(rbb.at[b], rb_hbm.at[pl.ds(base // 8, CH // 8)], so[b])
            pf = jnp.minimum(cc + 2, NCH - 1)
            issue_in(pf, b)
        return ()

    lax.fori_loop(0, NCH // 2, chunk, ())

    for b in range(2):
        g1, g2 = pend[b]
        g1.wait()
        g2.wait()
        pltpu.make_async_copy(shb.at[b], sh_hbm.at[pl.ds(0, CH)], so[b]).wait()
        pltpu.make_async_copy(rbb.at[b], rb_hbm.at[pl.ds(0, CH // 8)], so[b]).wait()

    for nc in range(NODES_W // 64):
        nb = wid * NODES_W + nc * 64
        pltpu.sync_copy(z_hbm.at[pl.ds(nb, 64)], zidx)
        pltpu.async_copy(emb_hbm.at[zidx], embr, sa0).wait()
        pltpu.sync_copy(embr, h0_hbm.at[pl.ds(nb, 64)])


def _geom_consts():
    ii = [[3, 0, 1, 2, 0, 1, 2, 0, 0] + [3] * 7,
          [3, 3, 3, 3, 1, 2, 2, 2, 0] + [3] * 7,
          [3] * 6 + [3, 3, 1] + [3] * 7,
          [3] * 6 + [3, 3, 1] + [3] * 7]
    ci = jnp.array(ii, jnp.int32)
    k1 = [1.0, _SQ3, _SQ3, _SQ3, _SQ15, _SQ15, 3.0 * _SQ5H, _SQ15, _SQ15H] + [0.0] * 7
    k2 = [0.0] * 6 + [-_SQ5H, 0.0, -_SQ15H] + [0.0] * 7
    oneh3 = [0.0] * 3 + [1.0] + [0.0] * 12
    steps = [r * _RBF_W for r in range(N_RBF)] + [1e6] * 6
    cf = jnp.array([k1, k2, oneh3, steps], jnp.float32)
    return ci, cf


NAGG = N_NODES


@functools.partial(
    pl.kernel,
    out_type=jax.ShapeDtypeStruct((NC, NAGG, MSGW), jnp.float32),
    mesh=_mesh,
    scratch_types=[
        pltpu.VMEM((NCH, CH), jnp.int32),
        pltpu.VMEM((NCH, CH), jnp.int32),
        pltpu.VMEM((2, CH, 16), jnp.float32),
        pltpu.VMEM((2, CH // 8, 128), jnp.float32),
        pltpu.VMEM((2, CH, 16), jnp.float32),
        pltpu.VMEM((2, CH, MSGW), jnp.float32),
        pltpu.VMEM_SHARED((NAGG, MSGW), jnp.float32),
        pltpu.SemaphoreType.DMA,
        pltpu.SemaphoreType.DMA,
        pltpu.SemaphoreType.DMA,
        pltpu.SemaphoreType.DMA,
        pltpu.SemaphoreType.DMA,
        pltpu.SemaphoreType.DMA,
    ],
    compiler_params=pltpu.CompilerParams(use_tc_tiling_on_sc=False),
)
def _sc_edge(src2_hbm, dst2_hbm, hw_hbm, rw_hbm, sh_hbm, out_hbm,
             sidx, didx, hwb, rwb, shb, msgb, agg_sh,
             sg0, sg1, sl0, sl1, ss0, ss1):
    cid = lax.axis_index("c")
    sid = lax.axis_index("s")
    zero16 = jnp.zeros((LANES,), jnp.float32)
    sg = (sg0, sg1)
    sl = (sl0, sl1)
    ss = (ss0, ss1)

    def zmsg(r, _):
        for cc in range(SH_DIM):
            msgb[0, r, pl.ds(cc * 16, 16)] = zero16
        return ()
    lax.fori_loop(0, CH, zmsg, ())
    for r in range(7):
        pltpu.sync_copy(msgb.at[0], agg_sh.at[pl.ds(sid * 625 + r * CH, CH)])
    pltpu.sync_copy(msgb.at[0].at[pl.ds(0, 65)],
                    agg_sh.at[pl.ds(sid * 625 + 7 * CH, 65)])
    plsc.subcore_barrier()

    wrow0 = (cid * NS + sid) * NCH
    pltpu.sync_copy(src2_hbm.at[pl.ds(wrow0, NCH)], sidx)
    pltpu.sync_copy(dst2_hbm.at[pl.ds(wrow0, NCH)], didx)

    def issue_in(ci, b):
        g = pltpu.async_copy(hw_hbm.at[sidx.at[ci]], hwb.at[b], sg[b])
        l1 = pltpu.async_copy(rw_hbm.at[pl.ds((wrow0 + ci) * (CH // 8), CH // 8)], rwb.at[b], sl[b])
        l2 = pltpu.async_copy(sh_hbm.at[pl.ds((wrow0 + ci) * CH, CH)], shb.at[b], sl[b])
        return g, l1, l2

    pend = [issue_in(0, 0), issue_in(1, 1)]

    def outer(go, _):
        for b in range(2):
            ci = go * 2 + b
            pltpu.make_async_copy(sh_hbm.at[pl.ds(0, CH)], hwb.at[b], sg[b]).wait()
            pltpu.make_async_copy(rw_hbm.at[pl.ds(0, CH // 8)], rwb.at[b], sl[b]).wait()
            pltpu.make_async_copy(sh_hbm.at[pl.ds(0, CH)], shb.at[b], sl[b]).wait()

            @pl.when(ci >= 2)
            def _():
                pltpu.make_async_copy(msgb.at[b], agg_sh.at[didx.at[0]],
                                      ss[b]).wait()

            def group(gg, _):
                for j in range(LANES):
                    i = gg * LANES + j
                    m = hwb[b, i] * rwb[b, 2 * gg + j // 8, pl.ds((j % 8) * 16, 16)]
                    shr = shb[b, i]
                    for k in range(SH_DIM):
                        sk = shr.at[jnp.full((LANES,), k, jnp.int32)].get(
                            mode='promise_in_bounds')
                        msgb[b, i, pl.ds(k * 16, 16)] = m * sk
                return ()
            lax.fori_loop(0, CH // LANES, group, ())
            pltpu.async_copy(msgb.at[b], agg_sh.at[didx.at[ci]], ss[b], add=True)
            pf = jnp.minimum(ci + 2, NCH - 1)
            issue_in(pf, b)
        return ()

    lax.fori_loop(0, NCH // 2, outer, ())

    for b in range(2):
        g, l1, l2 = pend[b]
        g.wait()
        l1.wait()
        l2.wait()
        pltpu.make_async_copy(msgb.at[b], agg_sh.at[didx.at[0]], ss[b]).wait()
    plsc.subcore_barrier()
    pltpu.sync_copy(agg_sh.at[pl.ds(sid * 625, 625)],
                    out_hbm.at[cid, pl.ds(sid * 625, 625)])


def _silu(x):
    return x / (1.0 + jnp.exp(-x))


def _tc_radial_body(rb_ref, bd1_ref, bd2_ref, out_ref):
    e = pl.program_id(0)
    rb = rb_ref[...]
    rows = lax.broadcasted_iota(jnp.int32, (256, 1), 0) + e * 256
    msk = (rows < N_EDGES * 16 // 128).astype(jnp.float32)
    t = _silu(jnp.dot(rb, bd1_ref[...], preferred_element_type=jnp.float32))
    tb = t.astype(jnp.bfloat16)
    for l in range(N_LAYERS):
        out_ref[l] = jnp.dot(tb, bd2_ref[l],
                             preferred_element_type=jnp.float32) * msk


def _tc_radial(rb, bd1, bd2):
    return pl.pallas_call(
        _tc_radial_body,
        grid=(E_PAD // 8 // 256,),
        in_specs=[
            pl.BlockSpec((256, 128), lambda i: (i, 0)),
            pl.BlockSpec((128, 1024), lambda i: (0, 0)),
            pl.BlockSpec((N_LAYERS, 1024, 128), lambda i: (0, 0, 0)),
        ],
        out_specs=pl.BlockSpec((N_LAYERS, 256, 128), lambda i: (0, i, 0)),
        out_shape=jax.ShapeDtypeStruct((N_LAYERS, E_PAD // 8, 128), jnp.float32),
    )(rb, bd1, bd2)


def _tc_hw_body(h_ref, w_ref, out_ref):
    out_ref[...] = jnp.dot(h_ref[...], w_ref[...],
                           preferred_element_type=jnp.float32)


def _tc_hw(h, w):
    n = h.shape[0]
    return pl.pallas_call(
        _tc_hw_body,
        grid=(n // 1000,),
        in_specs=[
            pl.BlockSpec((1000, h.shape[1]), lambda i: (i, 0)),
            pl.BlockSpec(w.shape, lambda i: (0, 0)),
        ],
        out_specs=pl.BlockSpec((1000, 16), lambda i: (i, 0)),
        out_shape=jax.ShapeDtypeStruct((n, 16), jnp.float32),
    )(h, w)


def _tc_node_body(agg_ref, h_ref, wout_ref, wsc_ref, wmsg_ref, h_o, hw_o):
    a = agg_ref[0] + agg_ref[1]
    hn = (jnp.dot(a, wout_ref[...], preferred_element_type=jnp.float32)
          + jnp.dot(h_ref[...], wsc_ref[...], preferred_element_type=jnp.float32))
    h_o[...] = hn
    hw_o[...] = jnp.dot(hn, wmsg_ref[...], preferred_element_type=jnp.float32)


def _tc_node(agg2, h, wout, wsc, wmsg):
    d_in = h.shape[1]
    return pl.pallas_call(
        _tc_node_body,
        grid=(N_NODES // 1000,),
        in_specs=[
            pl.BlockSpec((NC, 1000, MSGW), lambda i: (0, i, 0)),
            pl.BlockSpec((1000, d_in), lambda i: (i, 0)),
            pl.BlockSpec((MSGW, HID), lambda i: (0, 0)),
            pl.BlockSpec((d_in, HID), lambda i: (0, 0)),
            pl.BlockSpec((HID, 16), lambda i: (0, 0)),
        ],
        out_specs=[
            pl.BlockSpec((1000, HID), lambda i: (i, 0)),
            pl.BlockSpec((1000, 16), lambda i: (i, 0)),
        ],
        out_shape=[
            jax.ShapeDtypeStruct((N_NODES, HID), jnp.float32),
            jax.ShapeDtypeStruct((N_NODES, 16), jnp.float32),
        ],
    )(agg2, h, wout, wsc, wmsg)


def _tc_readout_body(h_ref, ai_ref, ga_ref, batch_ref,
                     wq_ref, wk_ref, wv_ref,
                     w1s_ref, w1c_ref, w1v_ref, w1t_ref, b1_ref,
                     w2_ref, b2_ref, s3_ref, s5_ref, out_ref, ha_ref):
    def gather_row(i, _):
        ha_ref[pl.ds(i, 1), :] = h_ref[pl.ds(ai_ref[i, 0], 1), :]
        return ()
    lax.fori_loop(0, N_GRAPHS, gather_row, ())
    scal = h_ref[:, :MUL0]
    k = jnp.dot(scal, wk_ref[...], preferred_element_type=jnp.float32)
    v = jnp.dot(scal, wv_ref[...], preferred_element_type=jnp.float32)
    sa = ha_ref[:, :MUL0]
    q = jnp.dot(sa, wq_ref[...], preferred_element_type=jnp.float32)
    scores = lax.dot_general(q, k, (((1,), (1,)), ((), ())),
                             preferred_element_type=jnp.float32)
    scores = scores * (1.0 / (MUL0 ** 0.5))
    valid = ga_ref[...] == batch_ref[...]
    scores = jnp.where(valid, scores, -1e9)
    mx = jnp.max(scores, axis=1, keepdims=True)
    e = jnp.exp(scores - mx)
    attn = e / jnp.sum(e, axis=1, keepdims=True)
    c = jnp.dot(attn, v, preferred_element_type=jnp.float32)
    vsq = ha_ref[:, MUL0:MUL0 + MUL1 * 3]
    nv = jnp.dot(vsq * vsq, s3_ref[...], preferred_element_type=jnp.float32)
    tsq = ha_ref[:, MUL0 + MUL1 * 3:HID]
    nt = jnp.dot(tsq * tsq, s5_ref[...], preferred_element_type=jnp.float32)
    zr = (jnp.dot(sa, w1s_ref[...], preferred_element_type=jnp.float32)
          + jnp.dot(c, w1c_ref[...], preferred_element_type=jnp.float32)
          + jnp.dot(nv, w1v_ref[...], preferred_element_type=jnp.float32)
          + jnp.dot(nt, w1t_ref[...], preferred_element_type=jnp.float32)
          + b1_ref[...])
    hdn = _silu(zr)
    out_ref[...] = jnp.dot(hdn, w2_ref[...],
                           preferred_element_type=jnp.float32) + b2_ref[...]


def _tc_readout(h, ai2, ga2, batch2, wq, wk, wv, w1s, w1c, w1v, w1t, b1, w2, b2,
                s3, s5):
    return pl.pallas_call(
        _tc_readout_body,
        out_shape=jax.ShapeDtypeStruct((N_GRAPHS, NUM_BASIS), jnp.float32),
        scratch_shapes=[pltpu.VMEM((N_GRAPHS, HID), jnp.float32)],
    )(h, ai2, ga2, batch2, wq, wk, wv, w1s, w1c, w1v, w1t, b1, w2, b2, s3, s5)


def kernel(z, pos, edge_index, batch, absorber_mask, params):
    f32 = jnp.float32
    z_pad = jnp.pad(z.astype(jnp.int32), (0, N_PAD - N_NODES))
    pos16 = jnp.zeros((N_PAD, 16), f32).at[:N_NODES, :3].set(pos)
    src = jnp.pad(edge_index[0].astype(jnp.int32), (0, E_PAD - N_EDGES))
    dst = jnp.pad(edge_index[1].astype(jnp.int32), (0, E_PAD - N_EDGES))
    abs_idx = jnp.nonzero(absorber_mask, size=N_GRAPHS)[0].astype(jnp.int32)
    g_a = batch[abs_idx].astype(jnp.int32)

    layers = params['layers']
    rw1cat = jnp.concatenate(
        [jnp.pad(lp['rw1'], ((0, 16 - N_RBF), (0, 0))) for lp in layers], axis=1)
    bd1 = jnp.zeros((128, 1024), f32)
    for e in range(8):
        bd1 = bd1.at[e * 16:(e + 1) * 16, e * 128:(e + 1) * 128].set(rw1cat)
    bd2 = jnp.zeros((N_LAYERS, 1024, 128), f32)
    for l in range(N_LAYERS):
        for e in range(8):
            bd2 = bd2.at[l, e * 128 + l * 32:e * 128 + (l + 1) * 32,
                         e * 16:(e + 1) * 16].set(layers[l]['rw2'])
    bd2 = bd2.astype(jnp.bfloat16)
    perm = (jnp.arange(MSGW) % 16) * SH_DIM + (jnp.arange(MSGW) // 16)
    wouts = [lp['w_out'][perm] * 0.25 for lp in layers]
    s3 = (jnp.arange(MUL1 * 3)[:, None] // 3 == jnp.arange(MUL1)[None, :]).astype(f32)
    s5 = (jnp.arange(MUL2 * 5)[:, None] // 5 == jnp.arange(MUL2)[None, :]).astype(f32)
    w1 = params['w1']
    w1s, w1c = w1[:MUL0], w1[MUL0:2 * MUL0]
    w1v, w1t = w1[2 * MUL0:2 * MUL0 + MUL1], w1[2 * MUL0 + MUL1:]

    ci_const, cf_const = _geom_consts()
    src2 = src.reshape(E_PAD // CH, CH)
    dst2 = dst.reshape(E_PAD // CH, CH)
    sh, rb, h0 = _sc_geom(pos16, src2, dst2, z_pad, params['emb'], ci_const, cf_const)
    rw_all = _tc_radial(rb, bd1, bd2)
    h = h0[:N_NODES]
    hw = _tc_hw(h, layers[0]['w_msg'])
    for l in range(N_LAYERS):
        agg2 = _sc_edge(src2, dst2, hw, rw_all[l], sh)
        wmsg_next = (layers[l + 1]['w_msg'] if l + 1 < N_LAYERS
                     else jnp.zeros((HID, 16), f32))
        h, hw = _tc_node(agg2, h, wouts[l], layers[l]['w_sc'], wmsg_next)

    return _tc_readout(h, abs_idx[:, None], g_a[:, None], batch.astype(jnp.int32)[None, :],
                       params['wq'], params['wk'], params['wv'],
                       w1s, w1c, w1v, w1t, params['b1'][None, :],
                       params['w2'], params['b2'][None, :], s3, s5)
